# Initial kernel scaffold; baseline (speedup 1.0000x reference)
import jax
import jax.numpy as jnp
from jax.experimental import pallas as pl


def kernel(x, pos_edge_index, neg_edge_index, W1, b1, W2, b2):
    raise NotImplementedError("write your pallas kernel here")


# speedup vs baseline: 3.6288x; 3.6288x over previous
import functools

import jax
import jax.numpy as jnp
from jax import lax
from jax.experimental import pallas as pl
from jax.experimental.pallas import tpu as pltpu
from jax.experimental.pallas import tpu_sc as plsc

N = 10000
D = 128
E = 320000
E2 = 2 * E

NC = 2
NS = 16
NPAD = 10240
NPT = NPAD // NS

CH = 80
EPC = E // NC
EPT = EPC // NS
NCHUNK = EPT // CH

E2PC = E2 // NC
E2PT = E2PC // NS
NCHUNK2 = E2PT // CH

_mesh = plsc.VectorSubcoreMesh(core_axis_name="c", subcore_axis_name="s")


NW = NC * NS


@functools.partial(
    pl.kernel,
    out_type=jax.ShapeDtypeStruct((NW * NPAD,), jnp.float32),
    mesh=_mesh,
    compiler_params=pltpu.CompilerParams(needs_layout_passes=False),
    scratch_types=[
        pltpu.VMEM((NPAD,), jnp.float32),
        pltpu.VMEM((EPT,), jnp.int32),
    ],
)
def _degree_sc(dst_hbm, out_hbm, hist, didx):
    c = lax.axis_index("c")
    s = lax.axis_index("s")

    def z(r, carry):
        for q in range(16):
            hist[pl.ds((r * 16 + q) * 16, 16)] = jnp.zeros((16,), jnp.float32)
        return carry

    lax.fori_loop(0, NPAD // 256, z, 0)

    base = c * EPC + s * EPT
    pltpu.sync_copy(dst_hbm.at[pl.ds(base, EPT)], didx)
    ones = jnp.ones((16,), jnp.float32)

    def step(t, carry):
        for q in range(25):
            iv = didx[pl.ds((t * 25 + q) * 16, 16)]
            plsc.addupdate_scatter(hist, [iv], ones)
        return carry

    lax.fori_loop(0, EPT // (25 * 16), step, 0)
    wid = c * NS + s
    pltpu.sync_copy(hist, out_hbm.at[pl.ds(wid * NPAD, NPAD)])


@functools.partial(
    pl.kernel,
    out_type=jax.ShapeDtypeStruct((NC, NPAD, D), jnp.float32),
    mesh=_mesh,
    compiler_params=pltpu.CompilerParams(needs_layout_passes=False),
    scratch_types=[
        pltpu.VMEM_SHARED((NPAD, D), jnp.float32),
        pltpu.VMEM((CH,), jnp.int32),
        pltpu.VMEM((CH,), jnp.int32),
        pltpu.VMEM((CH, D), jnp.float32),
        pltpu.SemaphoreType.DMA,
    ],
)
def _scatter_sc(table_hbm, src_hbm, dst_hbm, zeros_hbm, out_hbm, acc,
                sidx, didx, rows, sem):
    c = lax.axis_index("c")
    s = lax.axis_index("s")
    pltpu.sync_copy(zeros_hbm, acc.at[pl.ds(s * NPT, NPT)])
    plsc.subcore_barrier()
    base = c * EPC + s * EPT

    def chunk(t, carry):
        off = base + t * CH
        pltpu.sync_copy(src_hbm.at[pl.ds(off, CH)], sidx)
        pltpu.sync_copy(dst_hbm.at[pl.ds(off, CH)], didx)
        pltpu.async_copy(table_hbm.at[sidx], rows, sem).wait()
        pltpu.sync_copy(rows, acc.at[didx], add=True)
        return carry

    lax.fori_loop(0, NCHUNK, chunk, 0)
    plsc.subcore_barrier()
    pltpu.sync_copy(acc.at[pl.ds(s * NPT, NPT)],
                    out_hbm.at[c, pl.ds(s * NPT, NPT)])


@functools.partial(
    pl.kernel,
    out_type=jax.ShapeDtypeStruct((E2,), jnp.float32),
    mesh=_mesh,
    compiler_params=pltpu.CompilerParams(needs_layout_passes=False),
    scratch_types=[
        pltpu.VMEM((CH,), jnp.int32),
        pltpu.VMEM((CH,), jnp.int32),
        pltpu.VMEM((CH, D), jnp.float32),
        pltpu.VMEM((CH, D), jnp.float32),
        pltpu.VMEM((CH,), jnp.float32),
        pltpu.SemaphoreType.DMA,
        pltpu.SemaphoreType.DMA,
    ],
)
def _link_sc(h_hbm, ei_hbm, ej_hbm, out_hbm,
             iidx, jidx, rowsi, rowsj, dots, semi, semj):
    c = lax.axis_index("c")
    s = lax.axis_index("s")
    base = c * E2PC + s * E2PT
    iota = lax.iota(jnp.int32, 16)

    def chunk(t, carry):
        off = base + t * CH
        pltpu.sync_copy(ei_hbm.at[pl.ds(off, CH)], iidx)
        pltpu.sync_copy(ej_hbm.at[pl.ds(off, CH)], jidx)
        cpi = pltpu.async_copy(h_hbm.at[iidx], rowsi, semi)
        cpj = pltpu.async_copy(h_hbm.at[jidx], rowsj, semj)
        cpi.wait()
        cpj.wait()

        def group(g, carry2):
            rowid = g * 16 + iota
            accs = [jnp.zeros((16,), jnp.float32) for _ in range(4)]
            for f in range(D):
                col = jnp.full((16,), f, jnp.int32)
                vi = plsc.load_gather(rowsi, [rowid, col])
                vj = plsc.load_gather(rowsj, [rowid, col])
                accs[f % 4] = accs[f % 4] + vi * vj
            dots[pl.ds(g * 16, 16)] = (accs[0] + accs[1]) + (accs[2] + accs[3])
            return carry2

        lax.fori_loop(0, CH // 16, group, 0)
        pltpu.sync_copy(dots, out_hbm.at[pl.ds(off, CH)])
        return carry

    lax.fori_loop(0, NCHUNK2, chunk, 0)


def _dinv_from(deg_ref):
    deg = jnp.sum(deg_ref[...], axis=0)[:N]
    return lax.rsqrt(deg + 1.0)[:, None]


def _stage1_tc(deg_ref, x_ref, w1_ref, p1_ref):
    dinv = _dinv_from(deg_ref)
    g = jnp.dot(x_ref[...], w1_ref[...], preferred_element_type=jnp.float32,
                precision=lax.Precision.HIGHEST)
    p1_ref[...] = g * dinv


def _stage2_tc(deg_ref, s1_ref, p1_ref, b1_ref, w2_ref, p2_ref):
    dinv = _dinv_from(deg_ref)
    s1 = s1_ref[0][:N] + s1_ref[1][:N]
    h1 = jnp.maximum(dinv * (s1 + p1_ref[...]) + b1_ref[...], 0.0)
    g = jnp.dot(h1, w2_ref[...], preferred_element_type=jnp.float32,
                precision=lax.Precision.HIGHEST)
    p2_ref[...] = g * dinv


def _stage3_tc(deg_ref, s2_ref, p2_ref, b2_ref, h2_ref):
    dinv = _dinv_from(deg_ref)
    s2 = s2_ref[0][:N] + s2_ref[1][:N]
    h2_ref[...] = dinv * (s2 + p2_ref[...]) + b2_ref[...]


_f32 = jnp.float32


def kernel(x, pos_edge_index, neg_edge_index, W1, b1, W2, b2):
    src = pos_edge_index[0]
    dst = pos_edge_index[1]

    zeros_d = jnp.zeros((NPT, D), _f32)

    deg_part = _degree_sc(dst).reshape(NW, NPAD)

    p1 = pl.pallas_call(
        _stage1_tc, out_shape=jax.ShapeDtypeStruct((N, D), _f32),
    )(deg_part, x, W1)

    s1 = _scatter_sc(p1, src, dst, zeros_d)

    p2 = pl.pallas_call(
        _stage2_tc, out_shape=jax.ShapeDtypeStruct((N, D), _f32),
    )(deg_part, s1, p1, b1, W2)

    s2 = _scatter_sc(p2, src, dst, zeros_d)

    h2 = pl.pallas_call(
        _stage3_tc, out_shape=jax.ShapeDtypeStruct((N, D), _f32),
    )(deg_part, s2, p2, b2)

    ei = jnp.concatenate([pos_edge_index[0], neg_edge_index[0]])
    ej = jnp.concatenate([pos_edge_index[1], neg_edge_index[1]])
    return _link_sc(h2, ei, ej)


# speedup vs baseline: 4.6239x; 1.2742x over previous
import functools

import jax
import jax.numpy as jnp
from jax import lax
from jax.experimental import pallas as pl
from jax.experimental.pallas import tpu as pltpu
from jax.experimental.pallas import tpu_sc as plsc

N = 10000
D = 128
E = 320000
E2 = 2 * E

NC = 2
NS = 16
NPAD = 10240
NPT = NPAD // NS

CH = 80
EPC = E // NC
EPT = EPC // NS
NCHUNK = EPT // CH

E2PC = E2 // NC
E2PT = E2PC // NS
NCHUNK2 = E2PT // CH

_mesh = plsc.VectorSubcoreMesh(core_axis_name="c", subcore_axis_name="s")


NW = NC * NS


@functools.partial(
    pl.kernel,
    out_type=jax.ShapeDtypeStruct((NW * NPAD,), jnp.float32),
    mesh=_mesh,
    compiler_params=pltpu.CompilerParams(needs_layout_passes=False),
    scratch_types=[
        pltpu.VMEM((NPAD,), jnp.float32),
        pltpu.VMEM((EPT,), jnp.int32),
    ],
)
def _degree_sc(dst_hbm, out_hbm, hist, didx):
    c = lax.axis_index("c")
    s = lax.axis_index("s")

    def z(r, carry):
        for q in range(16):
            hist[pl.ds((r * 16 + q) * 16, 16)] = jnp.zeros((16,), jnp.float32)
        return carry

    lax.fori_loop(0, NPAD // 256, z, 0)

    base = c * EPC + s * EPT
    pltpu.sync_copy(dst_hbm.at[pl.ds(base, EPT)], didx)
    ones = jnp.ones((16,), jnp.float32)

    def step(t, carry):
        for q in range(25):
            iv = didx[pl.ds((t * 25 + q) * 16, 16)]
            plsc.addupdate_scatter(hist, [iv], ones)
        return carry

    lax.fori_loop(0, EPT // (25 * 16), step, 0)
    wid = c * NS + s
    pltpu.sync_copy(hist, out_hbm.at[pl.ds(wid * NPAD, NPAD)])


@functools.partial(
    pl.kernel,
    out_type=jax.ShapeDtypeStruct((NC, NPAD, D), jnp.float32),
    mesh=_mesh,
    compiler_params=pltpu.CompilerParams(needs_layout_passes=False),
    scratch_types=[
        pltpu.VMEM_SHARED((NPAD, D), jnp.float32),
        pltpu.VMEM((EPT,), jnp.int32),
        pltpu.VMEM((EPT,), jnp.int32),
        pltpu.VMEM((2, CH), jnp.int32),
        pltpu.VMEM((2, CH, D), jnp.float32),
        pltpu.SemaphoreType.DMA((2,)),
        pltpu.SemaphoreType.DMA((2,)),
    ],
)
def _scatter_sc(table_hbm, src_hbm, dst_hbm, zeros_hbm, out_hbm, acc,
                sidx, didx, dcur, rows, semg, sems):
    c = lax.axis_index("c")
    s = lax.axis_index("s")
    pltpu.sync_copy(zeros_hbm, acc.at[pl.ds(s * NPT, NPT)])
    base = c * EPC + s * EPT
    pltpu.sync_copy(src_hbm.at[pl.ds(base, EPT)], sidx)
    pltpu.sync_copy(dst_hbm.at[pl.ds(base, EPT)], didx)
    plsc.subcore_barrier()

    pltpu.async_copy(table_hbm.at[sidx.at[pl.ds(0, CH)]], rows.at[0],
                     semg.at[0])

    def chunk(t, carry):
        b = lax.rem(t, 2)
        nb = 1 - b
        pltpu.make_async_copy(table_hbm.at[sidx.at[pl.ds(t * CH, CH)]],
                              rows.at[b], semg.at[b]).wait()
        for q in range(CH // 16):
            dcur[b, pl.ds(q * 16, 16)] = didx[pl.ds(t * CH + q * 16, 16)]
        pltpu.async_copy(rows.at[b], acc.at[dcur.at[b]], sems.at[b], add=True)

        @pl.when(t + 1 < NCHUNK)
        def _():
            @pl.when(t >= 1)
            def _():
                pltpu.make_async_copy(rows.at[nb], acc.at[dcur.at[nb]],
                                      sems.at[nb]).wait()
            pltpu.async_copy(
                table_hbm.at[sidx.at[pl.ds((t + 1) * CH, CH)]],
                rows.at[nb], semg.at[nb])

        return carry

    lax.fori_loop(0, NCHUNK, chunk, 0)
    pltpu.make_async_copy(rows.at[1], acc.at[dcur.at[1]], sems.at[1]).wait()
    pltpu.make_async_copy(rows.at[0], acc.at[dcur.at[0]], sems.at[0]).wait()
    plsc.subcore_barrier()
    pltpu.sync_copy(acc.at[pl.ds(s * NPT, NPT)],
                    out_hbm.at[c, pl.ds(s * NPT, NPT)])


@functools.partial(
    pl.kernel,
    out_type=jax.ShapeDtypeStruct((E2,), jnp.float32),
    mesh=_mesh,
    compiler_params=pltpu.CompilerParams(needs_layout_passes=False),
    scratch_types=[
        pltpu.VMEM((E2PT,), jnp.int32),
        pltpu.VMEM((E2PT,), jnp.int32),
        pltpu.VMEM((2, CH, D), jnp.float32),
        pltpu.VMEM((2, CH, D), jnp.float32),
        pltpu.VMEM((E2PT,), jnp.float32),
        pltpu.SemaphoreType.DMA((2,)),
        pltpu.SemaphoreType.DMA((2,)),
    ],
)
def _link_sc(h_hbm, ei_hbm, ej_hbm, out_hbm,
             iidx, jidx, rowsi, rowsj, dots, semi, semj):
    c = lax.axis_index("c")
    s = lax.axis_index("s")
    base = c * E2PC + s * E2PT
    pltpu.sync_copy(ei_hbm.at[pl.ds(base, E2PT)], iidx)
    pltpu.sync_copy(ej_hbm.at[pl.ds(base, E2PT)], jidx)
    iota = lax.iota(jnp.int32, 16)

    pltpu.async_copy(h_hbm.at[iidx.at[pl.ds(0, CH)]], rowsi.at[0], semi.at[0])
    pltpu.async_copy(h_hbm.at[jidx.at[pl.ds(0, CH)]], rowsj.at[0], semj.at[0])

    def chunk(t, carry):
        b = lax.rem(t, 2)
        nb = 1 - b
        pltpu.make_async_copy(h_hbm.at[iidx.at[pl.ds(t * CH, CH)]],
                              rowsi.at[b], semi.at[b]).wait()
        pltpu.make_async_copy(h_hbm.at[jidx.at[pl.ds(t * CH, CH)]],
                              rowsj.at[b], semj.at[b]).wait()

        @pl.when(t + 1 < NCHUNK2)
        def _():
            pltpu.async_copy(h_hbm.at[iidx.at[pl.ds((t + 1) * CH, CH)]],
                             rowsi.at[nb], semi.at[nb])
            pltpu.async_copy(h_hbm.at[jidx.at[pl.ds((t + 1) * CH, CH)]],
                             rowsj.at[nb], semj.at[nb])

        bcol = jnp.full((16,), b, jnp.int32)

        def group(g, carry2):
            rowid = g * 16 + iota
            accs = [jnp.zeros((16,), jnp.float32) for _ in range(4)]
            for f in range(D):
                col = jnp.full((16,), f, jnp.int32)
                vi = plsc.load_gather(rowsi, [bcol, rowid, col])
                vj = plsc.load_gather(rowsj, [bcol, rowid, col])
                accs[f % 4] = accs[f % 4] + vi * vj
            dots[pl.ds(t * CH + g * 16, 16)] = (
                (accs[0] + accs[1]) + (accs[2] + accs[3]))
            return carry2

        lax.fori_loop(0, CH // 16, group, 0)
        return carry

    lax.fori_loop(0, NCHUNK2, chunk, 0)
    pltpu.sync_copy(dots, out_hbm.at[pl.ds(base, E2PT)])


def _dinv_from(deg_ref):
    deg = jnp.sum(deg_ref[...], axis=0)[:N]
    return lax.rsqrt(deg + 1.0)[:, None]


def _stage1_tc(deg_ref, x_ref, w1_ref, p1_ref):
    dinv = _dinv_from(deg_ref)
    g = jnp.dot(x_ref[...], w1_ref[...], preferred_element_type=jnp.float32,
                precision=lax.Precision.HIGHEST)
    p1_ref[...] = g * dinv


def _stage2_tc(deg_ref, s1_ref, p1_ref, b1_ref, w2_ref, p2_ref):
    dinv = _dinv_from(deg_ref)
    s1 = s1_ref[0][:N] + s1_ref[1][:N]
    h1 = jnp.maximum(dinv * (s1 + p1_ref[...]) + b1_ref[...], 0.0)
    g = jnp.dot(h1, w2_ref[...], preferred_element_type=jnp.float32,
                precision=lax.Precision.HIGHEST)
    p2_ref[...] = g * dinv


def _stage3_tc(deg_ref, s2_ref, p2_ref, b2_ref, h2_ref):
    dinv = _dinv_from(deg_ref)
    s2 = s2_ref[0][:N] + s2_ref[1][:N]
    h2_ref[...] = dinv * (s2 + p2_ref[...]) + b2_ref[...]


_f32 = jnp.float32


def kernel(x, pos_edge_index, neg_edge_index, W1, b1, W2, b2):
    src = pos_edge_index[0]
    dst = pos_edge_index[1]

    zeros_d = jnp.zeros((NPT, D), _f32)

    deg_part = _degree_sc(dst).reshape(NW, NPAD)

    p1 = pl.pallas_call(
        _stage1_tc, out_shape=jax.ShapeDtypeStruct((N, D), _f32),
    )(deg_part, x, W1)

    s1 = _scatter_sc(p1, src, dst, zeros_d)

    p2 = pl.pallas_call(
        _stage2_tc, out_shape=jax.ShapeDtypeStruct((N, D), _f32),
    )(deg_part, s1, p1, b1, W2)

    s2 = _scatter_sc(p2, src, dst, zeros_d)

    h2 = pl.pallas_call(
        _stage3_tc, out_shape=jax.ShapeDtypeStruct((N, D), _f32),
    )(deg_part, s2, p2, b2)

    ei = jnp.concatenate([pos_edge_index[0], neg_edge_index[0]])
    ej = jnp.concatenate([pos_edge_index[1], neg_edge_index[1]])
    return _link_sc(h2, ei, ej)


# speedup vs baseline: 14.0970x; 3.0487x over previous
import functools

import jax
import jax.numpy as jnp
from jax import lax
from jax.experimental import pallas as pl
from jax.experimental.pallas import tpu as pltpu
from jax.experimental.pallas import tpu_sc as plsc

N = 10000
D = 128
E = 320000
E2 = 2 * E

NC = 2
NS = 16
NPAD = 10240
NPT = NPAD // NS

CH = 80
EPC = E // NC
EPT = EPC // NS
NCHUNK = EPT // CH

E2PC = E2 // NC
E2PT = E2PC // NS
NCHUNK2 = E2PT // CH

_mesh = plsc.VectorSubcoreMesh(core_axis_name="c", subcore_axis_name="s")


NW = NC * NS


@functools.partial(
    pl.kernel,
    out_type=jax.ShapeDtypeStruct((NW * NPAD,), jnp.float32),
    mesh=_mesh,
    compiler_params=pltpu.CompilerParams(needs_layout_passes=False),
    scratch_types=[
        pltpu.VMEM((NPAD,), jnp.float32),
        pltpu.VMEM((EPT,), jnp.int32),
    ],
)
def _degree_sc(dst_hbm, out_hbm, hist, didx):
    c = lax.axis_index("c")
    s = lax.axis_index("s")

    def z(r, carry):
        for q in range(16):
            hist[pl.ds((r * 16 + q) * 16, 16)] = jnp.zeros((16,), jnp.float32)
        return carry

    lax.fori_loop(0, NPAD // 256, z, 0)

    base = c * EPC + s * EPT
    pltpu.sync_copy(dst_hbm.at[pl.ds(base, EPT)], didx)
    ones = jnp.ones((16,), jnp.float32)

    def step(t, carry):
        for q in range(25):
            iv = didx[pl.ds((t * 25 + q) * 16, 16)]
            plsc.addupdate_scatter(hist, [iv], ones)
        return carry

    lax.fori_loop(0, EPT // (25 * 16), step, 0)
    wid = c * NS + s
    pltpu.sync_copy(hist, out_hbm.at[pl.ds(wid * NPAD, NPAD)])


@functools.partial(
    pl.kernel,
    out_type=jax.ShapeDtypeStruct((NC, NPAD, D), jnp.float32),
    mesh=_mesh,
    compiler_params=pltpu.CompilerParams(needs_layout_passes=False),
    scratch_types=[
        pltpu.VMEM_SHARED((NPAD, D), jnp.float32),
        pltpu.VMEM((EPT,), jnp.int32),
        pltpu.VMEM((EPT,), jnp.int32),
        pltpu.VMEM((2, CH), jnp.int32),
        pltpu.VMEM((2, CH, D), jnp.float32),
        pltpu.SemaphoreType.DMA((2,)),
        pltpu.SemaphoreType.DMA((2,)),
    ],
)
def _scatter_sc(table_hbm, src_hbm, dst_hbm, zeros_hbm, out_hbm, acc,
                sidx, didx, dcur, rows, semg, sems):
    c = lax.axis_index("c")
    s = lax.axis_index("s")
    pltpu.sync_copy(zeros_hbm, acc.at[pl.ds(s * NPT, NPT)])
    base = c * EPC + s * EPT
    pltpu.sync_copy(src_hbm.at[pl.ds(base, EPT)], sidx)
    pltpu.sync_copy(dst_hbm.at[pl.ds(base, EPT)], didx)
    plsc.subcore_barrier()

    pltpu.async_copy(table_hbm.at[sidx.at[pl.ds(0, CH)]], rows.at[0],
                     semg.at[0])

    def chunk(t, carry):
        b = lax.rem(t, 2)
        nb = 1 - b
        pltpu.make_async_copy(table_hbm.at[sidx.at[pl.ds(t * CH, CH)]],
                              rows.at[b], semg.at[b]).wait()
        for q in range(CH // 16):
            dcur[b, pl.ds(q * 16, 16)] = didx[pl.ds(t * CH + q * 16, 16)]
        pltpu.async_copy(rows.at[b], acc.at[dcur.at[b]], sems.at[b], add=True)

        @pl.when(t + 1 < NCHUNK)
        def _():
            @pl.when(t >= 1)
            def _():
                pltpu.make_async_copy(rows.at[nb], acc.at[dcur.at[nb]],
                                      sems.at[nb]).wait()
            pltpu.async_copy(
                table_hbm.at[sidx.at[pl.ds((t + 1) * CH, CH)]],
                rows.at[nb], semg.at[nb])

        return carry

    lax.fori_loop(0, NCHUNK, chunk, 0)
    pltpu.make_async_copy(rows.at[1], acc.at[dcur.at[1]], sems.at[1]).wait()
    pltpu.make_async_copy(rows.at[0], acc.at[dcur.at[0]], sems.at[0]).wait()
    plsc.subcore_barrier()
    pltpu.sync_copy(acc.at[pl.ds(s * NPT, NPT)],
                    out_hbm.at[c, pl.ds(s * NPT, NPT)])


@functools.partial(
    pl.kernel,
    out_type=jax.ShapeDtypeStruct((E2,), jnp.float32),
    mesh=_mesh,
    compiler_params=pltpu.CompilerParams(needs_layout_passes=False),
    scratch_types=[
        pltpu.VMEM((E2PT,), jnp.int32),
        pltpu.VMEM((E2PT,), jnp.int32),
        pltpu.VMEM((2, CH, D), jnp.float32),
        pltpu.VMEM((2, CH, D), jnp.float32),
        pltpu.VMEM((E2PT,), jnp.float32),
        pltpu.SemaphoreType.DMA((2,)),
        pltpu.SemaphoreType.DMA((2,)),
    ],
)
def _link_sc(h_hbm, ei_hbm, ej_hbm, out_hbm,
             iidx, jidx, rowsi, rowsj, dots, semi, semj):
    c = lax.axis_index("c")
    s = lax.axis_index("s")
    base = c * E2PC + s * E2PT
    pltpu.sync_copy(ei_hbm.at[pl.ds(base, E2PT)], iidx)
    pltpu.sync_copy(ej_hbm.at[pl.ds(base, E2PT)], jidx)
    iota = lax.iota(jnp.int32, 16)

    pltpu.async_copy(h_hbm.at[iidx.at[pl.ds(0, CH)]], rowsi.at[0], semi.at[0])
    pltpu.async_copy(h_hbm.at[jidx.at[pl.ds(0, CH)]], rowsj.at[0], semj.at[0])

    def chunk(t, carry):
        b = lax.rem(t, 2)
        nb = 1 - b
        pltpu.make_async_copy(h_hbm.at[iidx.at[pl.ds(t * CH, CH)]],
                              rowsi.at[b], semi.at[b]).wait()
        pltpu.make_async_copy(h_hbm.at[jidx.at[pl.ds(t * CH, CH)]],
                              rowsj.at[b], semj.at[b]).wait()

        @pl.when(t + 1 < NCHUNK2)
        def _():
            pltpu.async_copy(h_hbm.at[iidx.at[pl.ds((t + 1) * CH, CH)]],
                             rowsi.at[nb], semi.at[nb])
            pltpu.async_copy(h_hbm.at[jidx.at[pl.ds((t + 1) * CH, CH)]],
                             rowsj.at[nb], semj.at[nb])

        def group(g, carry2):
            dvec = jnp.zeros((16,), jnp.float32)
            for e16 in range(16):
                e = g * 16 + e16
                a0 = jnp.zeros((16,), jnp.float32)
                a1 = jnp.zeros((16,), jnp.float32)
                for kk in range(D // 16):
                    vi = rowsi[b, e, pl.ds(kk * 16, 16)]
                    vj = rowsj[b, e, pl.ds(kk * 16, 16)]
                    if kk % 2 == 0:
                        a0 = a0 + vi * vj
                    else:
                        a1 = a1 + vi * vj
                dvec = jnp.where(iota == e16, jnp.sum(a0 + a1), dvec)
            dots[pl.ds(t * CH + g * 16, 16)] = dvec
            return carry2

        lax.fori_loop(0, CH // 16, group, 0)
        return carry

    lax.fori_loop(0, NCHUNK2, chunk, 0)
    pltpu.sync_copy(dots, out_hbm.at[pl.ds(base, E2PT)])


def _dinv_from(deg_ref):
    deg = jnp.sum(deg_ref[...], axis=0)[:N]
    return lax.rsqrt(deg + 1.0)[:, None]


def _stage1_tc(deg_ref, x_ref, w1_ref, p1_ref):
    dinv = _dinv_from(deg_ref)
    g = jnp.dot(x_ref[...], w1_ref[...], preferred_element_type=jnp.float32,
                precision=lax.Precision.HIGHEST)
    p1_ref[...] = g * dinv


def _stage2_tc(deg_ref, s1_ref, p1_ref, b1_ref, w2_ref, p2_ref):
    dinv = _dinv_from(deg_ref)
    s1 = s1_ref[0][:N] + s1_ref[1][:N]
    h1 = jnp.maximum(dinv * (s1 + p1_ref[...]) + b1_ref[...], 0.0)
    g = jnp.dot(h1, w2_ref[...], preferred_element_type=jnp.float32,
                precision=lax.Precision.HIGHEST)
    p2_ref[...] = g * dinv


def _stage3_tc(deg_ref, s2_ref, p2_ref, b2_ref, h2_ref):
    dinv = _dinv_from(deg_ref)
    s2 = s2_ref[0][:N] + s2_ref[1][:N]
    h2_ref[...] = dinv * (s2 + p2_ref[...]) + b2_ref[...]


_f32 = jnp.float32


def kernel(x, pos_edge_index, neg_edge_index, W1, b1, W2, b2):
    src = pos_edge_index[0]
    dst = pos_edge_index[1]

    zeros_d = jnp.zeros((NPT, D), _f32)

    deg_part = _degree_sc(dst).reshape(NW, NPAD)

    p1 = pl.pallas_call(
        _stage1_tc, out_shape=jax.ShapeDtypeStruct((N, D), _f32),
    )(deg_part, x, W1)

    s1 = _scatter_sc(p1, src, dst, zeros_d)

    p2 = pl.pallas_call(
        _stage2_tc, out_shape=jax.ShapeDtypeStruct((N, D), _f32),
    )(deg_part, s1, p1, b1, W2)

    s2 = _scatter_sc(p2, src, dst, zeros_d)

    h2 = pl.pallas_call(
        _stage3_tc, out_shape=jax.ShapeDtypeStruct((N, D), _f32),
    )(deg_part, s2, p2, b2)

    ei = jnp.concatenate([pos_edge_index[0], neg_edge_index[0]])
    ej = jnp.concatenate([pos_edge_index[1], neg_edge_index[1]])
    return _link_sc(h2, ei, ej)


# speedup vs baseline: 15.9529x; 1.1317x over previous
import functools

import jax
import jax.numpy as jnp
from jax import lax
from jax.experimental import pallas as pl
from jax.experimental.pallas import tpu as pltpu
from jax.experimental.pallas import tpu_sc as plsc

N = 10000
D = 128
E = 320000
E2 = 2 * E

NC = 2
NS = 16
NPAD = 10240
NPT = NPAD // NS

CH = 80
EPC = E // NC
EPT = EPC // NS
NCHUNK = EPT // CH

E2PC = E2 // NC
E2PT = E2PC // NS
NCHUNK2 = E2PT // CH

_mesh = plsc.VectorSubcoreMesh(core_axis_name="c", subcore_axis_name="s")


NW = NC * NS


@functools.partial(
    pl.kernel,
    out_type=jax.ShapeDtypeStruct((NW * NPAD,), jnp.float32),
    mesh=_mesh,
    compiler_params=pltpu.CompilerParams(needs_layout_passes=False),
    scratch_types=[
        pltpu.VMEM((NPAD,), jnp.float32),
        pltpu.VMEM((EPT,), jnp.int32),
    ],
)
def _degree_sc(dst_hbm, out_hbm, hist, didx):
    c = lax.axis_index("c")
    s = lax.axis_index("s")

    def z(r, carry):
        for q in range(16):
            hist[pl.ds((r * 16 + q) * 16, 16)] = jnp.zeros((16,), jnp.float32)
        return carry

    lax.fori_loop(0, NPAD // 256, z, 0)

    base = c * EPC + s * EPT
    pltpu.sync_copy(dst_hbm.at[pl.ds(base, EPT)], didx)
    ones = jnp.ones((16,), jnp.float32)

    def step(t, carry):
        for q in range(25):
            iv = didx[pl.ds((t * 25 + q) * 16, 16)]
            plsc.addupdate_scatter(hist, [iv], ones)
        return carry

    lax.fori_loop(0, EPT // (25 * 16), step, 0)
    wid = c * NS + s
    pltpu.sync_copy(hist, out_hbm.at[pl.ds(wid * NPAD, NPAD)])


@functools.partial(
    pl.kernel,
    out_type=jax.ShapeDtypeStruct((NC, NPAD, D), jnp.float32),
    mesh=_mesh,
    compiler_params=pltpu.CompilerParams(needs_layout_passes=False),
    scratch_types=[
        pltpu.VMEM_SHARED((NPAD, D), jnp.float32),
        pltpu.VMEM((EPT,), jnp.int32),
        pltpu.VMEM((EPT,), jnp.int32),
        pltpu.VMEM((2, CH), jnp.int32),
        pltpu.VMEM((2, CH, D), jnp.float32),
        pltpu.SemaphoreType.DMA((2,)),
        pltpu.SemaphoreType.DMA((2,)),
    ],
)
def _scatter_sc(table_hbm, src_hbm, dst_hbm, zeros_hbm, out_hbm, acc,
                sidx, didx, dcur, rows, semg, sems):
    c = lax.axis_index("c")
    s = lax.axis_index("s")
    pltpu.sync_copy(zeros_hbm, acc.at[pl.ds(s * NPT, NPT)])
    base = c * EPC + s * EPT
    pltpu.sync_copy(src_hbm.at[pl.ds(base, EPT)], sidx)
    pltpu.sync_copy(dst_hbm.at[pl.ds(base, EPT)], didx)
    plsc.subcore_barrier()

    pltpu.async_copy(table_hbm.at[sidx.at[pl.ds(0, CH)]], rows.at[0],
                     semg.at[0])

    def chunk(t, carry):
        b = lax.rem(t, 2)
        nb = 1 - b
        pltpu.make_async_copy(table_hbm.at[sidx.at[pl.ds(t * CH, CH)]],
                              rows.at[b], semg.at[b]).wait()
        for q in range(CH // 16):
            dcur[b, pl.ds(q * 16, 16)] = didx[pl.ds(t * CH + q * 16, 16)]
        pltpu.async_copy(rows.at[b], acc.at[dcur.at[b]], sems.at[b], add=True)

        @pl.when(t + 1 < NCHUNK)
        def _():
            @pl.when(t >= 1)
            def _():
                pltpu.make_async_copy(rows.at[nb], acc.at[dcur.at[nb]],
                                      sems.at[nb]).wait()
            pltpu.async_copy(
                table_hbm.at[sidx.at[pl.ds((t + 1) * CH, CH)]],
                rows.at[nb], semg.at[nb])

        return carry

    lax.fori_loop(0, NCHUNK, chunk, 0)
    pltpu.make_async_copy(rows.at[1], acc.at[dcur.at[1]], sems.at[1]).wait()
    pltpu.make_async_copy(rows.at[0], acc.at[dcur.at[0]], sems.at[0]).wait()
    plsc.subcore_barrier()
    pltpu.sync_copy(acc.at[pl.ds(s * NPT, NPT)],
                    out_hbm.at[c, pl.ds(s * NPT, NPT)])


DEPTH = 16


@functools.partial(
    pl.kernel,
    out_type=jax.ShapeDtypeStruct((E2,), jnp.float32),
    mesh=_mesh,
    compiler_params=pltpu.CompilerParams(needs_layout_passes=False),
    scratch_types=[
        pltpu.VMEM((E2PT,), jnp.int32),
        pltpu.VMEM((E2PT,), jnp.int32),
        pltpu.VMEM((E2PT,), jnp.int32),
        pltpu.VMEM((E2PT,), jnp.float32),
        pltpu.SemaphoreType.DMA,
    ],
)
def _link_sc(g_hbm, ei_hbm, ej_hbm, out_hbm, iidx, jidx, fidx, dots, sem):
    c = lax.axis_index("c")
    s = lax.axis_index("s")
    base = c * E2PC + s * E2PT
    pltpu.sync_copy(ei_hbm.at[pl.ds(base, E2PT)], iidx)
    pltpu.sync_copy(ej_hbm.at[pl.ds(base, E2PT)], jidx)

    def flat(t, carry):
        for q in range(5):
            o = (t * 5 + q) * 16
            fidx[pl.ds(o, 16)] = (iidx[pl.ds(o, 16)] * N
                                  + jidx[pl.ds(o, 16)])
        return carry

    lax.fori_loop(0, E2PT // 80, flat, 0)

    def fire(t, carry):
        pltpu.async_copy(g_hbm.at[fidx.at[pl.ds(t * CH, CH)]],
                         dots.at[pl.ds(t * CH, CH)], sem)

        @pl.when(t >= DEPTH)
        def _():
            pltpu.make_async_copy(g_hbm.at[fidx.at[pl.ds(0, CH)]],
                                  dots.at[pl.ds(0, CH)], sem).wait()

        return carry

    lax.fori_loop(0, NCHUNK2, fire, 0)
    for _q in range(DEPTH):
        pltpu.make_async_copy(g_hbm.at[fidx.at[pl.ds(0, CH)]],
                              dots.at[pl.ds(0, CH)], sem).wait()
    pltpu.sync_copy(dots, out_hbm.at[pl.ds(base, E2PT)])


GB = 400


def _gram_tc(h_ref, ht_ref, g_ref):
    a = h_ref[...].astype(jnp.bfloat16)
    b = ht_ref[...].astype(jnp.bfloat16)
    g_ref[...] = lax.dot_general(a, b, (((1,), (1,)), ((), ())),
                                 preferred_element_type=jnp.float32)


def _dinv_from(deg_ref):
    deg = jnp.sum(deg_ref[...], axis=0)[:N]
    return lax.rsqrt(deg + 1.0)[:, None]


def _stage1_tc(deg_ref, x_ref, w1_ref, p1_ref):
    dinv = _dinv_from(deg_ref)
    g = jnp.dot(x_ref[...], w1_ref[...], preferred_element_type=jnp.float32,
                precision=lax.Precision.HIGHEST)
    p1_ref[...] = g * dinv


def _stage2_tc(deg_ref, s1_ref, p1_ref, b1_ref, w2_ref, p2_ref):
    dinv = _dinv_from(deg_ref)
    s1 = s1_ref[0][:N] + s1_ref[1][:N]
    h1 = jnp.maximum(dinv * (s1 + p1_ref[...]) + b1_ref[...], 0.0)
    g = jnp.dot(h1, w2_ref[...], preferred_element_type=jnp.float32,
                precision=lax.Precision.HIGHEST)
    p2_ref[...] = g * dinv


def _stage3_tc(deg_ref, s2_ref, p2_ref, b2_ref, h2_ref):
    dinv = _dinv_from(deg_ref)
    s2 = s2_ref[0][:N] + s2_ref[1][:N]
    h2_ref[...] = dinv * (s2 + p2_ref[...]) + b2_ref[...]


_f32 = jnp.float32


def kernel(x, pos_edge_index, neg_edge_index, W1, b1, W2, b2):
    src = pos_edge_index[0]
    dst = pos_edge_index[1]

    zeros_d = jnp.zeros((NPT, D), _f32)

    deg_part = _degree_sc(dst).reshape(NW, NPAD)

    p1 = pl.pallas_call(
        _stage1_tc, out_shape=jax.ShapeDtypeStruct((N, D), _f32),
    )(deg_part, x, W1)

    s1 = _scatter_sc(p1, src, dst, zeros_d)

    p2 = pl.pallas_call(
        _stage2_tc, out_shape=jax.ShapeDtypeStruct((N, D), _f32),
    )(deg_part, s1, p1, b1, W2)

    s2 = _scatter_sc(p2, src, dst, zeros_d)

    h2 = pl.pallas_call(
        _stage3_tc, out_shape=jax.ShapeDtypeStruct((N, D), _f32),
    )(deg_part, s2, p2, b2)

    gram = pl.pallas_call(
        _gram_tc,
        grid=(N // GB,),
        in_specs=[
            pl.BlockSpec((GB, D), lambda i: (i, 0)),
            pl.BlockSpec((N, D), lambda i: (0, 0)),
        ],
        out_specs=pl.BlockSpec((GB, N), lambda i: (i, 0)),
        out_shape=jax.ShapeDtypeStruct((N, N), _f32),
    )(h2, h2)

    ei = jnp.concatenate([pos_edge_index[0], neg_edge_index[0]])
    ej = jnp.concatenate([pos_edge_index[1], neg_edge_index[1]])
    return _link_sc(gram.reshape(-1), ei, ej)


# speedup vs baseline: 15.9571x; 1.0003x over previous
import functools

import jax
import jax.numpy as jnp
from jax import lax
from jax.experimental import pallas as pl
from jax.experimental.pallas import tpu as pltpu
from jax.experimental.pallas import tpu_sc as plsc

N = 10000
D = 128
E = 320000
E2 = 2 * E

NC = 2
NS = 16
NPAD = 10240
NPT = NPAD // NS

CH = 80
EPC = E // NC
EPT = EPC // NS
NCHUNK = EPT // CH

E2PC = E2 // NC
E2PT = E2PC // NS
NCHUNK2 = E2PT // CH

_mesh = plsc.VectorSubcoreMesh(core_axis_name="c", subcore_axis_name="s")


NW = NC * NS


@functools.partial(
    pl.kernel,
    out_type=jax.ShapeDtypeStruct((NW * NPAD,), jnp.float32),
    mesh=_mesh,
    compiler_params=pltpu.CompilerParams(needs_layout_passes=False),
    scratch_types=[
        pltpu.VMEM((NPAD,), jnp.float32),
        pltpu.VMEM((EPT,), jnp.int32),
    ],
)
def _degree_sc(dst_hbm, out_hbm, hist, didx):
    c = lax.axis_index("c")
    s = lax.axis_index("s")

    def z(r, carry):
        for q in range(16):
            hist[pl.ds((r * 16 + q) * 16, 16)] = jnp.zeros((16,), jnp.float32)
        return carry

    lax.fori_loop(0, NPAD // 256, z, 0)

    base = c * EPC + s * EPT
    pltpu.sync_copy(dst_hbm.at[pl.ds(base, EPT)], didx)
    ones = jnp.ones((16,), jnp.float32)

    def step(t, carry):
        for q in range(25):
            iv = didx[pl.ds((t * 25 + q) * 16, 16)]
            plsc.addupdate_scatter(hist, [iv], ones)
        return carry

    lax.fori_loop(0, EPT // (25 * 16), step, 0)
    wid = c * NS + s
    pltpu.sync_copy(hist, out_hbm.at[pl.ds(wid * NPAD, NPAD)])


@functools.partial(
    pl.kernel,
    out_type=jax.ShapeDtypeStruct((NC, NPAD, D), jnp.float32),
    mesh=_mesh,
    compiler_params=pltpu.CompilerParams(needs_layout_passes=False),
    scratch_types=[
        pltpu.VMEM_SHARED((NPAD, D), jnp.float32),
        pltpu.VMEM((EPT,), jnp.int32),
        pltpu.VMEM((EPT,), jnp.int32),
        pltpu.VMEM((2, CH), jnp.int32),
        pltpu.VMEM((2, CH, D), jnp.float32),
        pltpu.SemaphoreType.DMA((2,)),
        pltpu.SemaphoreType.DMA((2,)),
    ],
)
def _scatter_sc(table_hbm, src_hbm, dst_hbm, zeros_hbm, out_hbm, acc,
                sidx, didx, dcur, rows, semg, sems):
    c = lax.axis_index("c")
    s = lax.axis_index("s")
    pltpu.sync_copy(zeros_hbm, acc.at[pl.ds(s * NPT, NPT)])
    base = c * EPC + s * EPT
    pltpu.sync_copy(src_hbm.at[pl.ds(base, EPT)], sidx)
    pltpu.sync_copy(dst_hbm.at[pl.ds(base, EPT)], didx)
    plsc.subcore_barrier()

    pltpu.async_copy(table_hbm.at[sidx.at[pl.ds(0, CH)]], rows.at[0],
                     semg.at[0])

    def chunk(t, carry):
        b = lax.rem(t, 2)
        nb = 1 - b
        pltpu.make_async_copy(table_hbm.at[sidx.at[pl.ds(t * CH, CH)]],
                              rows.at[b], semg.at[b]).wait()
        for q in range(CH // 16):
            dcur[b, pl.ds(q * 16, 16)] = didx[pl.ds(t * CH + q * 16, 16)]
        pltpu.async_copy(rows.at[b], acc.at[dcur.at[b]], sems.at[b], add=True)

        @pl.when(t + 1 < NCHUNK)
        def _():
            @pl.when(t >= 1)
            def _():
                pltpu.make_async_copy(rows.at[nb], acc.at[dcur.at[nb]],
                                      sems.at[nb]).wait()
            pltpu.async_copy(
                table_hbm.at[sidx.at[pl.ds((t + 1) * CH, CH)]],
                rows.at[nb], semg.at[nb])

        return carry

    lax.fori_loop(0, NCHUNK, chunk, 0)
    pltpu.make_async_copy(rows.at[1], acc.at[dcur.at[1]], sems.at[1]).wait()
    pltpu.make_async_copy(rows.at[0], acc.at[dcur.at[0]], sems.at[0]).wait()
    plsc.subcore_barrier()
    pltpu.sync_copy(acc.at[pl.ds(s * NPT, NPT)],
                    out_hbm.at[c, pl.ds(s * NPT, NPT)])


DEPTH = 16


@functools.partial(
    pl.kernel,
    out_type=jax.ShapeDtypeStruct((E2,), jnp.float32),
    mesh=_mesh,
    compiler_params=pltpu.CompilerParams(needs_layout_passes=False),
    scratch_types=[
        pltpu.VMEM((E2PT,), jnp.int32),
        pltpu.VMEM((E2PT,), jnp.int32),
        pltpu.VMEM((E2PT,), jnp.int32),
        pltpu.VMEM((E2PT,), jnp.float32),
        pltpu.SemaphoreType.DMA,
    ],
)
def _link_sc(g_hbm, ei_hbm, ej_hbm, out_hbm, iidx, jidx, fidx, dots, sem):
    c = lax.axis_index("c")
    s = lax.axis_index("s")
    base = c * E2PC + s * E2PT
    pltpu.sync_copy(ei_hbm.at[pl.ds(base, E2PT)], iidx)
    pltpu.sync_copy(ej_hbm.at[pl.ds(base, E2PT)], jidx)

    def flat(t, carry):
        for q in range(5):
            o = (t * 5 + q) * 16
            fidx[pl.ds(o, 16)] = (iidx[pl.ds(o, 16)] * N
                                  + jidx[pl.ds(o, 16)])
        return carry

    lax.fori_loop(0, E2PT // 80, flat, 0)

    def fire(t, carry):
        pltpu.async_copy(g_hbm.at[fidx.at[pl.ds(t * CH, CH)]],
                         dots.at[pl.ds(t * CH, CH)], sem)

        @pl.when(t >= DEPTH)
        def _():
            pltpu.make_async_copy(g_hbm.at[fidx.at[pl.ds(0, CH)]],
                                  dots.at[pl.ds(0, CH)], sem).wait()

        return carry

    lax.fori_loop(0, NCHUNK2, fire, 0)
    for _q in range(DEPTH):
        pltpu.make_async_copy(g_hbm.at[fidx.at[pl.ds(0, CH)]],
                              dots.at[pl.ds(0, CH)], sem).wait()
    pltpu.sync_copy(dots, out_hbm.at[pl.ds(base, E2PT)])


GB = 200


def _gram_tc(h_ref, ht_ref, g_ref):
    a = h_ref[...].astype(jnp.bfloat16)
    b = ht_ref[...].astype(jnp.bfloat16)
    g_ref[...] = lax.dot_general(a, b, (((1,), (1,)), ((), ())),
                                 preferred_element_type=jnp.float32)


def _dinv_from(deg_ref):
    deg = jnp.sum(deg_ref[...], axis=0)[:N]
    return lax.rsqrt(deg + 1.0)[:, None]


def _stage1_tc(deg_ref, x_ref, w1_ref, p1_ref):
    dinv = _dinv_from(deg_ref)
    g = jnp.dot(x_ref[...], w1_ref[...], preferred_element_type=jnp.float32,
                precision=lax.Precision.HIGHEST)
    p1_ref[...] = g * dinv


def _stage2_tc(deg_ref, s1_ref, p1_ref, b1_ref, w2_ref, p2_ref):
    dinv = _dinv_from(deg_ref)
    s1 = s1_ref[0][:N] + s1_ref[1][:N]
    h1 = jnp.maximum(dinv * (s1 + p1_ref[...]) + b1_ref[...], 0.0)
    g = jnp.dot(h1, w2_ref[...], preferred_element_type=jnp.float32,
                precision=lax.Precision.HIGHEST)
    p2_ref[...] = g * dinv


def _stage3_tc(deg_ref, s2_ref, p2_ref, b2_ref, h2_ref):
    dinv = _dinv_from(deg_ref)
    s2 = s2_ref[0][:N] + s2_ref[1][:N]
    h2_ref[...] = dinv * (s2 + p2_ref[...]) + b2_ref[...]


_f32 = jnp.float32


def kernel(x, pos_edge_index, neg_edge_index, W1, b1, W2, b2):
    src = pos_edge_index[0]
    dst = pos_edge_index[1]

    zeros_d = jnp.zeros((NPT, D), _f32)

    deg_part = _degree_sc(dst).reshape(NW, NPAD)

    p1 = pl.pallas_call(
        _stage1_tc, out_shape=jax.ShapeDtypeStruct((N, D), _f32),
    )(deg_part, x, W1)

    s1 = _scatter_sc(p1, src, dst, zeros_d)

    p2 = pl.pallas_call(
        _stage2_tc, out_shape=jax.ShapeDtypeStruct((N, D), _f32),
    )(deg_part, s1, p1, b1, W2)

    s2 = _scatter_sc(p2, src, dst, zeros_d)

    h2 = pl.pallas_call(
        _stage3_tc, out_shape=jax.ShapeDtypeStruct((N, D), _f32),
    )(deg_part, s2, p2, b2)

    gram = pl.pallas_call(
        _gram_tc,
        grid=(N // GB,),
        in_specs=[
            pl.BlockSpec((GB, D), lambda i: (i, 0)),
            pl.BlockSpec((N, D), lambda i: (0, 0)),
        ],
        out_specs=pl.BlockSpec((GB, N), lambda i: (i, 0)),
        out_shape=jax.ShapeDtypeStruct((N, N), _f32),
    )(h2, h2)

    ei = jnp.concatenate([pos_edge_index[0], neg_edge_index[0]])
    ej = jnp.concatenate([pos_edge_index[1], neg_edge_index[1]])
    return _link_sc(gram.reshape(-1), ei, ej)


# speedup vs baseline: 25.9762x; 1.6279x over previous
import functools

import jax
import jax.numpy as jnp
from jax import lax
from jax.experimental import pallas as pl
from jax.experimental.pallas import tpu as pltpu
from jax.experimental.pallas import tpu_sc as plsc

N = 10000
D = 128
E = 320000
E2 = 2 * E

NC = 2
NS = 16
NPAD = 10240
NPT = NPAD // NS

CH = 80
EPC = E // NC
EPT = EPC // NS
NCHUNK = EPT // CH

E2PC = E2 // NC
E2PT = E2PC // NS
NCHUNK2 = E2PT // CH

_mesh = plsc.VectorSubcoreMesh(core_axis_name="c", subcore_axis_name="s")


NW = NC * NS


@functools.partial(
    pl.kernel,
    out_type=jax.ShapeDtypeStruct((NW * NPAD,), jnp.float32),
    mesh=_mesh,
    compiler_params=pltpu.CompilerParams(needs_layout_passes=False),
    scratch_types=[
        pltpu.VMEM((NPAD,), jnp.float32),
        pltpu.VMEM((EPT,), jnp.int32),
    ],
)
def _degree_sc(dst_hbm, out_hbm, hist, didx):
    c = lax.axis_index("c")
    s = lax.axis_index("s")

    def z(r, carry):
        for q in range(16):
            hist[pl.ds((r * 16 + q) * 16, 16)] = jnp.zeros((16,), jnp.float32)
        return carry

    lax.fori_loop(0, NPAD // 256, z, 0)

    base = c * EPC + s * EPT
    pltpu.sync_copy(dst_hbm.at[pl.ds(base, EPT)], didx)
    ones = jnp.ones((16,), jnp.float32)

    def step(t, carry):
        for q in range(25):
            iv = didx[pl.ds((t * 25 + q) * 16, 16)]
            plsc.addupdate_scatter(hist, [iv], ones)
        return carry

    lax.fori_loop(0, EPT // (25 * 16), step, 0)
    wid = c * NS + s
    pltpu.sync_copy(hist, out_hbm.at[pl.ds(wid * NPAD, NPAD)])


@functools.partial(
    pl.kernel,
    out_type=jax.ShapeDtypeStruct((NC, NPAD, D), jnp.float32),
    mesh=_mesh,
    compiler_params=pltpu.CompilerParams(needs_layout_passes=False),
    scratch_types=[
        pltpu.VMEM_SHARED((NPAD, D), jnp.float32),
        pltpu.VMEM((EPT,), jnp.int32),
        pltpu.VMEM((EPT,), jnp.int32),
        pltpu.VMEM((2, CH), jnp.int32),
        pltpu.VMEM((2, CH, D), jnp.float32),
        pltpu.SemaphoreType.DMA((2,)),
        pltpu.SemaphoreType.DMA((2,)),
    ],
)
def _scatter_sc(table_hbm, src_hbm, dst_hbm, zeros_hbm, out_hbm, acc,
                sidx, didx, dcur, rows, semg, sems):
    c = lax.axis_index("c")
    s = lax.axis_index("s")
    pltpu.sync_copy(zeros_hbm, acc.at[pl.ds(s * NPT, NPT)])
    base = c * EPC + s * EPT
    pltpu.sync_copy(src_hbm.at[pl.ds(base, EPT)], sidx)
    pltpu.sync_copy(dst_hbm.at[pl.ds(base, EPT)], didx)
    plsc.subcore_barrier()

    pltpu.async_copy(table_hbm.at[sidx.at[pl.ds(0, CH)]], rows.at[0],
                     semg.at[0])

    def chunk(t, carry):
        b = lax.rem(t, 2)
        nb = 1 - b
        pltpu.make_async_copy(table_hbm.at[sidx.at[pl.ds(t * CH, CH)]],
                              rows.at[b], semg.at[b]).wait()
        for q in range(CH // 16):
            dcur[b, pl.ds(q * 16, 16)] = didx[pl.ds(t * CH + q * 16, 16)]
        pltpu.async_copy(rows.at[b], acc.at[dcur.at[b]], sems.at[b], add=True)

        @pl.when(t + 1 < NCHUNK)
        def _():
            @pl.when(t >= 1)
            def _():
                pltpu.make_async_copy(rows.at[nb], acc.at[dcur.at[nb]],
                                      sems.at[nb]).wait()
            pltpu.async_copy(
                table_hbm.at[sidx.at[pl.ds((t + 1) * CH, CH)]],
                rows.at[nb], semg.at[nb])

        return carry

    lax.fori_loop(0, NCHUNK, chunk, 0)
    pltpu.make_async_copy(rows.at[1], acc.at[dcur.at[1]], sems.at[1]).wait()
    pltpu.make_async_copy(rows.at[0], acc.at[dcur.at[0]], sems.at[0]).wait()
    plsc.subcore_barrier()
    pltpu.sync_copy(acc.at[pl.ds(s * NPT, NPT)],
                    out_hbm.at[c, pl.ds(s * NPT, NPT)])


DEPTH = 16


@functools.partial(
    pl.kernel,
    out_type=jax.ShapeDtypeStruct((E2,), jnp.float32),
    mesh=_mesh,
    compiler_params=pltpu.CompilerParams(needs_layout_passes=False),
    scratch_types=[
        pltpu.VMEM((E2PT,), jnp.int32),
        pltpu.VMEM((E2PT,), jnp.int32),
        pltpu.VMEM((E2PT,), jnp.int32),
        pltpu.VMEM((E2PT,), jnp.float32),
        pltpu.SemaphoreType.DMA,
    ],
)
def _link_sc(g_hbm, ei_hbm, ej_hbm, out_hbm, iidx, jidx, fidx, dots, sem):
    c = lax.axis_index("c")
    s = lax.axis_index("s")
    base = c * E2PC + s * E2PT
    pltpu.sync_copy(ei_hbm.at[pl.ds(base, E2PT)], iidx)
    pltpu.sync_copy(ej_hbm.at[pl.ds(base, E2PT)], jidx)

    def flat(t, carry):
        for q in range(5):
            o = (t * 5 + q) * 16
            iv = iidx[pl.ds(o, 16)]
            jv = jidx[pl.ds(o, 16)]
            jb = lax.shift_right_logical(jv, 7)
            cc = jv & 127
            fidx[pl.ds(o, 16)] = jb * (NPAD * D) + iv * D + cc
        return carry

    lax.fori_loop(0, E2PT // 80, flat, 0)

    def fire(t, carry):
        pltpu.async_copy(g_hbm.at[fidx.at[pl.ds(t * CH, CH)]],
                         dots.at[pl.ds(t * CH, CH)], sem)

        @pl.when(t >= DEPTH)
        def _():
            pltpu.make_async_copy(g_hbm.at[fidx.at[pl.ds(0, CH)]],
                                  dots.at[pl.ds(0, CH)], sem).wait()

        return carry

    lax.fori_loop(0, NCHUNK2, fire, 0)
    for _q in range(DEPTH):
        pltpu.make_async_copy(g_hbm.at[fidx.at[pl.ds(0, CH)]],
                              dots.at[pl.ds(0, CH)], sem).wait()
    pltpu.sync_copy(dots, out_hbm.at[pl.ds(base, E2PT)])


NJB = NPAD // D


def _gram_tc(h_ref, hs_ref, g_ref):
    a = h_ref[...].astype(jnp.bfloat16)
    b = hs_ref[...].astype(jnp.bfloat16)
    g_ref[...] = lax.dot_general(a, b, (((1,), (1,)), ((), ())),
                                 preferred_element_type=jnp.float32)[None]


def _dinv_from(deg_ref):
    deg = jnp.sum(deg_ref[...], axis=0)[:N]
    return lax.rsqrt(deg + 1.0)[:, None]


def _stage1_tc(deg_ref, x_ref, w1_ref, p1_ref):
    dinv = _dinv_from(deg_ref)
    g = jnp.dot(x_ref[...], w1_ref[...], preferred_element_type=jnp.float32,
                precision=lax.Precision.HIGHEST)
    p1_ref[...] = g * dinv


def _stage2_tc(deg_ref, s1_ref, p1_ref, b1_ref, w2_ref, p2_ref):
    dinv = _dinv_from(deg_ref)
    s1 = s1_ref[0][:N] + s1_ref[1][:N]
    h1 = jnp.maximum(dinv * (s1 + p1_ref[...]) + b1_ref[...], 0.0)
    g = jnp.dot(h1, w2_ref[...], preferred_element_type=jnp.float32,
                precision=lax.Precision.HIGHEST)
    p2_ref[...] = g * dinv


def _stage3_tc(deg_ref, s2_ref, p2_ref, b2_ref, h2_ref):
    dinv = _dinv_from(deg_ref)
    s2 = s2_ref[0][:N] + s2_ref[1][:N]
    h2_ref[...] = jnp.concatenate(
        [dinv * (s2 + p2_ref[...]) + b2_ref[...],
         jnp.zeros((NPAD - N, D), jnp.float32)], axis=0)


_f32 = jnp.float32


def kernel(x, pos_edge_index, neg_edge_index, W1, b1, W2, b2):
    src = pos_edge_index[0]
    dst = pos_edge_index[1]

    zeros_d = jnp.zeros((NPT, D), _f32)

    deg_part = _degree_sc(dst).reshape(NW, NPAD)

    p1 = pl.pallas_call(
        _stage1_tc, out_shape=jax.ShapeDtypeStruct((N, D), _f32),
    )(deg_part, x, W1)

    s1 = _scatter_sc(p1, src, dst, zeros_d)

    p2 = pl.pallas_call(
        _stage2_tc, out_shape=jax.ShapeDtypeStruct((N, D), _f32),
    )(deg_part, s1, p1, b1, W2)

    s2 = _scatter_sc(p2, src, dst, zeros_d)

    h2 = pl.pallas_call(
        _stage3_tc, out_shape=jax.ShapeDtypeStruct((NPAD, D), _f32),
    )(deg_part, s2, p2, b2)

    gram = pl.pallas_call(
        _gram_tc,
        grid=(NJB,),
        in_specs=[
            pl.BlockSpec((NPAD, D), lambda j: (0, 0)),
            pl.BlockSpec((D, D), lambda j: (j, 0)),
        ],
        out_specs=pl.BlockSpec((1, NPAD, D), lambda j: (j, 0, 0)),
        out_shape=jax.ShapeDtypeStruct((NJB, NPAD, D), _f32),
    )(h2, h2)

    ei = jnp.concatenate([pos_edge_index[0], neg_edge_index[0]])
    ej = jnp.concatenate([pos_edge_index[1], neg_edge_index[1]])
    return _link_sc(gram.reshape(-1), ei, ej)


# speedup vs baseline: 32.8623x; 1.2651x over previous
import functools

import jax
import jax.numpy as jnp
from jax import lax
from jax.experimental import pallas as pl
from jax.experimental.pallas import tpu as pltpu
from jax.experimental.pallas import tpu_sc as plsc

N = 10000
D = 128
E = 320000
E2 = 2 * E

NC = 2
NS = 16
NPAD = 10240
NPT = NPAD // NS

CH = 80
EPC = E // NC
EPT = EPC // NS
NCHUNK = EPT // CH

E2PC = E2 // NC
E2PT = E2PC // NS
NCHUNK2 = E2PT // CH

_mesh = plsc.VectorSubcoreMesh(core_axis_name="c", subcore_axis_name="s")


NW = NC * NS


@functools.partial(
    pl.kernel,
    out_type=jax.ShapeDtypeStruct((NW * NPAD,), jnp.float32),
    mesh=_mesh,
    compiler_params=pltpu.CompilerParams(needs_layout_passes=False),
    scratch_types=[
        pltpu.VMEM((NPAD,), jnp.float32),
        pltpu.VMEM((EPT,), jnp.int32),
    ],
)
def _degree_sc(dst_hbm, out_hbm, hist, didx):
    c = lax.axis_index("c")
    s = lax.axis_index("s")

    def z(r, carry):
        for q in range(16):
            hist[pl.ds((r * 16 + q) * 16, 16)] = jnp.zeros((16,), jnp.float32)
        return carry

    lax.fori_loop(0, NPAD // 256, z, 0)

    base = c * EPC + s * EPT
    pltpu.sync_copy(dst_hbm.at[pl.ds(base, EPT)], didx)
    ones = jnp.ones((16,), jnp.float32)

    def step(t, carry):
        for q in range(25):
            iv = didx[pl.ds((t * 25 + q) * 16, 16)]
            plsc.addupdate_scatter(hist, [iv], ones)
        return carry

    lax.fori_loop(0, EPT // (25 * 16), step, 0)
    wid = c * NS + s
    pltpu.sync_copy(hist, out_hbm.at[pl.ds(wid * NPAD, NPAD)])


NBUF = 3


@functools.partial(
    pl.kernel,
    out_type=jax.ShapeDtypeStruct((NC, NPAD, D), jnp.float32),
    mesh=_mesh,
    compiler_params=pltpu.CompilerParams(needs_layout_passes=False),
    scratch_types=[
        pltpu.VMEM_SHARED((NPAD, D), jnp.float32),
        pltpu.VMEM((EPT,), jnp.int32),
        pltpu.VMEM((NBUF, CH), jnp.int32),
        pltpu.VMEM((NBUF, CH, D), jnp.float32),
        pltpu.SemaphoreType.DMA((NBUF,)),
        pltpu.SemaphoreType.DMA((NBUF,)),
        pltpu.SemaphoreType.DMA((NBUF,)),
    ],
)
def _scatter_sc(table_hbm, src_hbm, dst_hbm, zeros_hbm, out_hbm, acc,
                sidx, dcur, rows, semg, semd, sems):
    c = lax.axis_index("c")
    s = lax.axis_index("s")
    pltpu.sync_copy(zeros_hbm, acc.at[pl.ds(s * NPT, NPT)])
    base = c * EPC + s * EPT
    pltpu.sync_copy(src_hbm.at[pl.ds(base, EPT)], sidx)
    plsc.subcore_barrier()

    for p in range(NBUF - 1):
        pltpu.async_copy(table_hbm.at[sidx.at[pl.ds(p * CH, CH)]],
                         rows.at[p], semg.at[p])
        pltpu.async_copy(dst_hbm.at[pl.ds(base + p * CH, CH)],
                         dcur.at[p], semd.at[p])

    def chunk(t, carry):
        b = lax.rem(t, NBUF)
        nb = lax.rem(t + NBUF - 1, NBUF)
        pltpu.make_async_copy(table_hbm.at[sidx.at[pl.ds(t * CH, CH)]],
                              rows.at[b], semg.at[b]).wait()
        pltpu.make_async_copy(dst_hbm.at[pl.ds(base, CH)],
                              dcur.at[b], semd.at[b]).wait()
        pltpu.async_copy(rows.at[b], acc.at[dcur.at[b]], sems.at[b], add=True)

        @pl.when(t + NBUF - 1 < NCHUNK)
        def _():
            @pl.when(t >= 1)
            def _():
                pltpu.make_async_copy(rows.at[nb], acc.at[dcur.at[nb]],
                                      sems.at[nb]).wait()
            pltpu.async_copy(
                table_hbm.at[sidx.at[pl.ds((t + NBUF - 1) * CH, CH)]],
                rows.at[nb], semg.at[nb])
            pltpu.async_copy(
                dst_hbm.at[pl.ds(base + (t + NBUF - 1) * CH, CH)],
                dcur.at[nb], semd.at[nb])

        return carry

    lax.fori_loop(0, NCHUNK, chunk, 0)
    for p in range(NBUF):
        pltpu.make_async_copy(rows.at[p], acc.at[dcur.at[p]],
                              sems.at[p]).wait()
    plsc.subcore_barrier()
    pltpu.sync_copy(acc.at[pl.ds(s * NPT, NPT)],
                    out_hbm.at[c, pl.ds(s * NPT, NPT)])


DEPTH = 16


@functools.partial(
    pl.kernel,
    out_type=jax.ShapeDtypeStruct((E2,), jnp.float32),
    mesh=_mesh,
    compiler_params=pltpu.CompilerParams(needs_layout_passes=False),
    scratch_types=[
        pltpu.VMEM((E2PT,), jnp.int32),
        pltpu.VMEM((E2PT,), jnp.int32),
        pltpu.VMEM((E2PT,), jnp.int32),
        pltpu.VMEM((E2PT,), jnp.float32),
        pltpu.SemaphoreType.DMA,
    ],
)
def _link_sc(g_hbm, ei_hbm, ej_hbm, out_hbm, iidx, jidx, fidx, dots, sem):
    c = lax.axis_index("c")
    s = lax.axis_index("s")
    base = c * E2PC + s * E2PT
    pltpu.sync_copy(ei_hbm.at[pl.ds(base, E2PT)], iidx)
    pltpu.sync_copy(ej_hbm.at[pl.ds(base, E2PT)], jidx)

    def flat(t, carry):
        for q in range(5):
            o = (t * 5 + q) * 16
            iv = iidx[pl.ds(o, 16)]
            jv = jidx[pl.ds(o, 16)]
            jb = lax.shift_right_logical(jv, 7)
            cc = jv & 127
            fidx[pl.ds(o, 16)] = jb * (NPAD * D) + iv * D + cc
        return carry

    lax.fori_loop(0, E2PT // 80, flat, 0)

    def fire(t, carry):
        pltpu.async_copy(g_hbm.at[fidx.at[pl.ds(t * CH, CH)]],
                         dots.at[pl.ds(t * CH, CH)], sem)

        @pl.when(t >= DEPTH)
        def _():
            pltpu.make_async_copy(g_hbm.at[fidx.at[pl.ds(0, CH)]],
                                  dots.at[pl.ds(0, CH)], sem).wait()

        return carry

    lax.fori_loop(0, NCHUNK2, fire, 0)
    for _q in range(DEPTH):
        pltpu.make_async_copy(g_hbm.at[fidx.at[pl.ds(0, CH)]],
                              dots.at[pl.ds(0, CH)], sem).wait()
    pltpu.sync_copy(dots, out_hbm.at[pl.ds(base, E2PT)])


NJB = NPAD // D


def _gram_tc(h_ref, hs_ref, g_ref):
    a = h_ref[...].astype(jnp.bfloat16)
    b = hs_ref[...].astype(jnp.bfloat16)
    g_ref[...] = lax.dot_general(a, b, (((1,), (1,)), ((), ())),
                                 preferred_element_type=jnp.float32)[None]


def _dinv_from(deg_ref):
    deg = jnp.sum(deg_ref[...], axis=0)[:N]
    return lax.rsqrt(deg + 1.0)[:, None]


def _stage1_tc(deg_ref, x_ref, w1_ref, p1_ref):
    dinv = _dinv_from(deg_ref)
    g = jnp.dot(x_ref[...], w1_ref[...], preferred_element_type=jnp.float32,
                precision=lax.Precision.HIGHEST)
    p1_ref[...] = g * dinv


def _stage2_tc(deg_ref, s1_ref, p1_ref, b1_ref, w2_ref, p2_ref):
    dinv = _dinv_from(deg_ref)
    s1 = s1_ref[0][:N] + s1_ref[1][:N]
    h1 = jnp.maximum(dinv * (s1 + p1_ref[...]) + b1_ref[...], 0.0)
    g = jnp.dot(h1, w2_ref[...], preferred_element_type=jnp.float32,
                precision=lax.Precision.HIGHEST)
    p2_ref[...] = g * dinv


def _stage3_tc(deg_ref, s2_ref, p2_ref, b2_ref, h2_ref):
    dinv = _dinv_from(deg_ref)
    s2 = s2_ref[0][:N] + s2_ref[1][:N]
    h2_ref[...] = jnp.concatenate(
        [dinv * (s2 + p2_ref[...]) + b2_ref[...],
         jnp.zeros((NPAD - N, D), jnp.float32)], axis=0)


_f32 = jnp.float32


def kernel(x, pos_edge_index, neg_edge_index, W1, b1, W2, b2):
    src = pos_edge_index[0]
    dst = pos_edge_index[1]

    zeros_d = jnp.zeros((NPT, D), _f32)

    deg_part = _degree_sc(dst).reshape(NW, NPAD)

    p1 = pl.pallas_call(
        _stage1_tc, out_shape=jax.ShapeDtypeStruct((N, D), _f32),
    )(deg_part, x, W1)

    s1 = _scatter_sc(p1, src, dst, zeros_d)

    p2 = pl.pallas_call(
        _stage2_tc, out_shape=jax.ShapeDtypeStruct((N, D), _f32),
    )(deg_part, s1, p1, b1, W2)

    s2 = _scatter_sc(p2, src, dst, zeros_d)

    h2 = pl.pallas_call(
        _stage3_tc, out_shape=jax.ShapeDtypeStruct((NPAD, D), _f32),
    )(deg_part, s2, p2, b2)

    gram = pl.pallas_call(
        _gram_tc,
        grid=(NJB,),
        in_specs=[
            pl.BlockSpec((NPAD, D), lambda j: (0, 0)),
            pl.BlockSpec((D, D), lambda j: (j, 0)),
        ],
        out_specs=pl.BlockSpec((1, NPAD, D), lambda j: (j, 0, 0)),
        out_shape=jax.ShapeDtypeStruct((NJB, NPAD, D), _f32),
    )(h2, h2)

    ei = jnp.concatenate([pos_edge_index[0], neg_edge_index[0]])
    ej = jnp.concatenate([pos_edge_index[1], neg_edge_index[1]])
    return _link_sc(gram.reshape(-1), ei, ej)


# speedup vs baseline: 33.2968x; 1.0132x over previous
import functools

import jax
import jax.numpy as jnp
from jax import lax
from jax.experimental import pallas as pl
from jax.experimental.pallas import tpu as pltpu
from jax.experimental.pallas import tpu_sc as plsc

N = 10000
D = 128
E = 320000
E2 = 2 * E

NC = 2
NS = 16
NPAD = 10240
NPT = NPAD // NS

CH = 80
EPC = E // NC
EPT = EPC // NS
NCHUNK = EPT // CH

E2PC = E2 // NC
E2PT = E2PC // NS
NCHUNK2 = E2PT // CH

_mesh = plsc.VectorSubcoreMesh(core_axis_name="c", subcore_axis_name="s")


NW = NC * NS


@functools.partial(
    pl.kernel,
    out_type=jax.ShapeDtypeStruct((NW * NPAD,), jnp.float32),
    mesh=_mesh,
    compiler_params=pltpu.CompilerParams(needs_layout_passes=False),
    scratch_types=[
        pltpu.VMEM((NPAD,), jnp.float32),
        pltpu.VMEM((EPT,), jnp.int32),
    ],
)
def _degree_sc(dst_hbm, out_hbm, hist, didx):
    c = lax.axis_index("c")
    s = lax.axis_index("s")

    def z(r, carry):
        for q in range(16):
            hist[pl.ds((r * 16 + q) * 16, 16)] = jnp.zeros((16,), jnp.float32)
        return carry

    lax.fori_loop(0, NPAD // 256, z, 0)

    base = c * EPC + s * EPT
    pltpu.sync_copy(dst_hbm.at[pl.ds(base, EPT)], didx)
    ones = jnp.ones((16,), jnp.float32)

    def step(t, carry):
        for q in range(25):
            iv = didx[pl.ds((t * 25 + q) * 16, 16)]
            plsc.addupdate_scatter(hist, [iv], ones)
        return carry

    lax.fori_loop(0, EPT // (25 * 16), step, 0)
    wid = c * NS + s
    pltpu.sync_copy(hist, out_hbm.at[pl.ds(wid * NPAD, NPAD)])


NBUF = 3


@functools.partial(
    pl.kernel,
    out_type=jax.ShapeDtypeStruct((NC, NPAD, D), jnp.float32),
    mesh=_mesh,
    compiler_params=pltpu.CompilerParams(needs_layout_passes=False),
    scratch_types=[
        pltpu.VMEM_SHARED((NPAD, D), jnp.float32),
        pltpu.VMEM((EPT,), jnp.int32),
        pltpu.VMEM((NBUF, CH), jnp.int32),
        pltpu.VMEM((NBUF, CH, D), jnp.float32),
        pltpu.SemaphoreType.DMA((NBUF,)),
        pltpu.SemaphoreType.DMA((NBUF,)),
        pltpu.SemaphoreType.DMA((NBUF,)),
    ],
)
def _scatter_sc(table_hbm, src_hbm, dst_hbm, zeros_hbm, out_hbm, acc,
                sidx, dcur, rows, semg, semd, sems):
    c = lax.axis_index("c")
    s = lax.axis_index("s")
    pltpu.sync_copy(zeros_hbm, acc.at[pl.ds(s * NPT, NPT)])
    base = c * EPC + s * EPT
    pltpu.sync_copy(src_hbm.at[pl.ds(base, EPT)], sidx)
    plsc.subcore_barrier()

    for p in range(NBUF - 1):
        pltpu.async_copy(table_hbm.at[sidx.at[pl.ds(p * CH, CH)]],
                         rows.at[p], semg.at[p])
        pltpu.async_copy(dst_hbm.at[pl.ds(base + p * CH, CH)],
                         dcur.at[p], semd.at[p])

    def chunk(t, carry):
        b = lax.rem(t, NBUF)
        nb = lax.rem(t + NBUF - 1, NBUF)
        pltpu.make_async_copy(table_hbm.at[sidx.at[pl.ds(t * CH, CH)]],
                              rows.at[b], semg.at[b]).wait()
        pltpu.make_async_copy(dst_hbm.at[pl.ds(base, CH)],
                              dcur.at[b], semd.at[b]).wait()
        pltpu.async_copy(rows.at[b], acc.at[dcur.at[b]], sems.at[b], add=True)

        @pl.when(t + NBUF - 1 < NCHUNK)
        def _():
            @pl.when(t >= 1)
            def _():
                pltpu.make_async_copy(rows.at[nb], acc.at[dcur.at[nb]],
                                      sems.at[nb]).wait()
            pltpu.async_copy(
                table_hbm.at[sidx.at[pl.ds((t + NBUF - 1) * CH, CH)]],
                rows.at[nb], semg.at[nb])
            pltpu.async_copy(
                dst_hbm.at[pl.ds(base + (t + NBUF - 1) * CH, CH)],
                dcur.at[nb], semd.at[nb])

        return carry

    lax.fori_loop(0, NCHUNK, chunk, 0)
    for p in range(NBUF):
        pltpu.make_async_copy(rows.at[p], acc.at[dcur.at[p]],
                              sems.at[p]).wait()
    plsc.subcore_barrier()
    pltpu.sync_copy(acc.at[pl.ds(s * NPT, NPT)],
                    out_hbm.at[c, pl.ds(s * NPT, NPT)])


DEPTH = 16


@functools.partial(
    pl.kernel,
    out_type=jax.ShapeDtypeStruct((E2,), jnp.float32),
    mesh=_mesh,
    compiler_params=pltpu.CompilerParams(needs_layout_passes=False),
    scratch_types=[
        pltpu.VMEM((E2PT,), jnp.int32),
        pltpu.VMEM((E2PT,), jnp.int32),
        pltpu.VMEM((E2PT,), jnp.int32),
        pltpu.VMEM((E2PT,), jnp.float32),
        pltpu.SemaphoreType.DMA,
    ],
)
def _link_sc(g_hbm, ei_hbm, ej_hbm, out_hbm, iidx, jidx, fidx, dots, sem):
    c = lax.axis_index("c")
    s = lax.axis_index("s")
    base = c * E2PC + s * E2PT
    pltpu.sync_copy(ei_hbm.at[pl.ds(base, E2PT)], iidx)
    pltpu.sync_copy(ej_hbm.at[pl.ds(base, E2PT)], jidx)

    def flat(t, carry):
        for q in range(5):
            o = (t * 5 + q) * 16
            iv = iidx[pl.ds(o, 16)]
            jv = jidx[pl.ds(o, 16)]
            jb = lax.shift_right_logical(jv, 7)
            cc = jv & 127
            fidx[pl.ds(o, 16)] = jb * (NPAD * D) + iv * D + cc
        return carry

    lax.fori_loop(0, E2PT // 80, flat, 0)

    def fire(t, carry):
        pltpu.async_copy(g_hbm.at[fidx.at[pl.ds(t * CH, CH)]],
                         dots.at[pl.ds(t * CH, CH)], sem)

        @pl.when(t >= DEPTH)
        def _():
            pltpu.make_async_copy(g_hbm.at[fidx.at[pl.ds(0, CH)]],
                                  dots.at[pl.ds(0, CH)], sem).wait()

        return carry

    lax.fori_loop(0, NCHUNK2, fire, 0)
    for _q in range(DEPTH):
        pltpu.make_async_copy(g_hbm.at[fidx.at[pl.ds(0, CH)]],
                              dots.at[pl.ds(0, CH)], sem).wait()
    pltpu.sync_copy(dots, out_hbm.at[pl.ds(base, E2PT)])


NJB = NPAD // D


def _dinv_from(deg_ref):
    deg = jnp.sum(deg_ref[...], axis=0)[:N]
    return lax.rsqrt(deg + 1.0)[:, None]


def _stage1_tc(deg_ref, x_ref, w1_ref, p1_ref):
    dinv = _dinv_from(deg_ref)
    g = jnp.dot(x_ref[...], w1_ref[...], preferred_element_type=jnp.float32,
                precision=lax.Precision.HIGHEST)
    p1_ref[...] = g * dinv


def _stage2_tc(deg_ref, s1_ref, p1_ref, b1_ref, w2_ref, p2_ref):
    dinv = _dinv_from(deg_ref)
    s1 = s1_ref[0][:N] + s1_ref[1][:N]
    h1 = jnp.maximum(dinv * (s1 + p1_ref[...]) + b1_ref[...], 0.0)
    g = jnp.dot(h1, w2_ref[...], preferred_element_type=jnp.float32,
                precision=lax.Precision.HIGHEST)
    p2_ref[...] = g * dinv


def _gram_fused_tc(deg_ref, s2_ref, p2_ref, b2_ref, g_ref, h2b):
    @pl.when(pl.program_id(0) == 0)
    def _():
        dinv = _dinv_from(deg_ref)
        s2 = s2_ref[0][:N] + s2_ref[1][:N]
        h2 = dinv * (s2 + p2_ref[...]) + b2_ref[...]
        h2b[...] = jnp.concatenate(
            [h2, jnp.zeros((NPAD - N, D), jnp.float32)],
            axis=0).astype(jnp.bfloat16)

    j = pl.program_id(0)
    a = h2b[...]
    b = h2b[pl.ds(j * D, D), :]
    g_ref[...] = lax.dot_general(a, b, (((1,), (1,)), ((), ())),
                                 preferred_element_type=jnp.float32)[None]


_f32 = jnp.float32


def kernel(x, pos_edge_index, neg_edge_index, W1, b1, W2, b2):
    src = pos_edge_index[0]
    dst = pos_edge_index[1]

    zeros_d = jnp.zeros((NPT, D), _f32)

    deg_part = _degree_sc(dst).reshape(NW, NPAD)

    p1 = pl.pallas_call(
        _stage1_tc, out_shape=jax.ShapeDtypeStruct((N, D), _f32),
    )(deg_part, x, W1)

    s1 = _scatter_sc(p1, src, dst, zeros_d)

    p2 = pl.pallas_call(
        _stage2_tc, out_shape=jax.ShapeDtypeStruct((N, D), _f32),
    )(deg_part, s1, p1, b1, W2)

    s2 = _scatter_sc(p2, src, dst, zeros_d)

    gram = pl.pallas_call(
        _gram_fused_tc,
        grid=(NJB,),
        in_specs=[
            pl.BlockSpec((NW, NPAD), lambda j: (0, 0)),
            pl.BlockSpec((NC, NPAD, D), lambda j: (0, 0, 0)),
            pl.BlockSpec((N, D), lambda j: (0, 0)),
            pl.BlockSpec((D,), lambda j: (0,)),
        ],
        out_specs=pl.BlockSpec((1, NPAD, D), lambda j: (j, 0, 0)),
        out_shape=jax.ShapeDtypeStruct((NJB, NPAD, D), _f32),
        scratch_shapes=[pltpu.VMEM((NPAD, D), jnp.bfloat16)],
    )(deg_part, s2, p2, b2)

    ei = jnp.concatenate([pos_edge_index[0], neg_edge_index[0]])
    ej = jnp.concatenate([pos_edge_index[1], neg_edge_index[1]])
    return _link_sc(gram.reshape(-1), ei, ej)


# speedup vs baseline: 36.0200x; 1.0818x over previous
import functools

import jax
import jax.numpy as jnp
from jax import lax
from jax.experimental import pallas as pl
from jax.experimental.pallas import tpu as pltpu
from jax.experimental.pallas import tpu_sc as plsc

N = 10000
D = 128
E = 320000
E2 = 2 * E

NC = 2
NS = 16
NPAD = 10240
NPT = NPAD // NS

CH = 80
EPC = E // NC
EPT = EPC // NS
NCHUNK = EPT // CH

E2PC = E2 // NC
E2PT = E2PC // NS
NCHUNK2 = E2PT // CH

_mesh = plsc.VectorSubcoreMesh(core_axis_name="c", subcore_axis_name="s")


NW = NC * NS


@functools.partial(
    pl.kernel,
    out_type=jax.ShapeDtypeStruct((NW * NPAD,), jnp.float32),
    mesh=_mesh,
    compiler_params=pltpu.CompilerParams(needs_layout_passes=False),
    scratch_types=[
        pltpu.VMEM((NPAD,), jnp.float32),
        pltpu.VMEM((EPT,), jnp.int32),
    ],
)
def _degree_sc(dst_hbm, out_hbm, hist, didx):
    c = lax.axis_index("c")
    s = lax.axis_index("s")

    def z(r, carry):
        for q in range(16):
            hist[pl.ds((r * 16 + q) * 16, 16)] = jnp.zeros((16,), jnp.float32)
        return carry

    lax.fori_loop(0, NPAD // 256, z, 0)

    base = c * EPC + s * EPT
    pltpu.sync_copy(dst_hbm.at[pl.ds(base, EPT)], didx)
    ones = jnp.ones((16,), jnp.float32)

    def step(t, carry):
        for q in range(25):
            iv = didx[pl.ds((t * 25 + q) * 16, 16)]
            plsc.addupdate_scatter(hist, [iv], ones)
        return carry

    lax.fori_loop(0, EPT // (25 * 16), step, 0)
    wid = c * NS + s
    pltpu.sync_copy(hist, out_hbm.at[pl.ds(wid * NPAD, NPAD)])


NBUF = 3


@functools.partial(
    pl.kernel,
    out_type=jax.ShapeDtypeStruct((NC, NPAD, D), jnp.float32),
    mesh=_mesh,
    compiler_params=pltpu.CompilerParams(needs_layout_passes=False),
    scratch_types=[
        pltpu.VMEM_SHARED((NPAD, D), jnp.float32),
        pltpu.VMEM((EPT,), jnp.int32),
        pltpu.VMEM((NBUF, CH), jnp.int32),
        pltpu.VMEM((NBUF, CH, D), jnp.float32),
        pltpu.SemaphoreType.DMA((NBUF,)),
        pltpu.SemaphoreType.DMA((NBUF,)),
        pltpu.SemaphoreType.DMA((NBUF,)),
    ],
)
def _scatter_sc(table_hbm, src_hbm, dst_hbm, zeros_hbm, out_hbm, acc,
                sidx, dcur, rows, semg, semd, sems):
    c = lax.axis_index("c")
    s = lax.axis_index("s")
    pltpu.sync_copy(zeros_hbm, acc.at[pl.ds(s * NPT, NPT)])
    base = c * EPC + s * EPT
    pltpu.sync_copy(src_hbm.at[pl.ds(base, EPT)], sidx)
    plsc.subcore_barrier()

    for p in range(NBUF - 1):
        pltpu.async_copy(table_hbm.at[sidx.at[pl.ds(p * CH, CH)]],
                         rows.at[p], semg.at[p])
        pltpu.async_copy(dst_hbm.at[pl.ds(base + p * CH, CH)],
                         dcur.at[p], semd.at[p])

    def chunk(t, carry):
        b = lax.rem(t, NBUF)
        nb = lax.rem(t + NBUF - 1, NBUF)
        pltpu.make_async_copy(table_hbm.at[sidx.at[pl.ds(t * CH, CH)]],
                              rows.at[b], semg.at[b]).wait()
        pltpu.make_async_copy(dst_hbm.at[pl.ds(base, CH)],
                              dcur.at[b], semd.at[b]).wait()
        pltpu.async_copy(rows.at[b], acc.at[dcur.at[b]], sems.at[b], add=True)

        @pl.when(t + NBUF - 1 < NCHUNK)
        def _():
            @pl.when(t >= 1)
            def _():
                pltpu.make_async_copy(rows.at[nb], acc.at[dcur.at[nb]],
                                      sems.at[nb]).wait()
            pltpu.async_copy(
                table_hbm.at[sidx.at[pl.ds((t + NBUF - 1) * CH, CH)]],
                rows.at[nb], semg.at[nb])
            pltpu.async_copy(
                dst_hbm.at[pl.ds(base + (t + NBUF - 1) * CH, CH)],
                dcur.at[nb], semd.at[nb])

        return carry

    lax.fori_loop(0, NCHUNK, chunk, 0)
    for p in range(NBUF):
        pltpu.make_async_copy(rows.at[p], acc.at[dcur.at[p]],
                              sems.at[p]).wait()
    plsc.subcore_barrier()
    pltpu.sync_copy(acc.at[pl.ds(s * NPT, NPT)],
                    out_hbm.at[c, pl.ds(s * NPT, NPT)])


DEPTH = 16


@functools.partial(
    pl.kernel,
    out_type=jax.ShapeDtypeStruct((E2,), jnp.float32),
    mesh=_mesh,
    compiler_params=pltpu.CompilerParams(needs_layout_passes=False),
    scratch_types=[
        pltpu.VMEM((E2PT,), jnp.int32),
        pltpu.VMEM((E2PT,), jnp.int32),
        pltpu.VMEM((E2PT,), jnp.int32),
        pltpu.VMEM((E2PT,), jnp.float32),
        pltpu.SemaphoreType.DMA,
    ],
)
def _link_sc(g_hbm, ei_hbm, ej_hbm, out_hbm, iidx, jidx, fidx, dots, sem):
    c = lax.axis_index("c")
    s = lax.axis_index("s")
    base = c * E2PC + s * E2PT
    pltpu.sync_copy(ei_hbm.at[pl.ds(base, E2PT)], iidx)
    pltpu.sync_copy(ej_hbm.at[pl.ds(base, E2PT)], jidx)

    def flat(t, carry):
        for q in range(5):
            o = (t * 5 + q) * 16
            iv = iidx[pl.ds(o, 16)]
            jv = jidx[pl.ds(o, 16)]
            jb = lax.shift_right_logical(jv, 8)
            cc = jv & 127
            fidx[pl.ds(o, 16)] = jb * (NPAD * D) + iv * D + cc
        return carry

    lax.fori_loop(0, E2PT // 80, flat, 0)

    def fire(t, carry):
        pltpu.async_copy(g_hbm.at[fidx.at[pl.ds(t * CH, CH)]],
                         dots.at[pl.ds(t * CH, CH)], sem)

        @pl.when(t >= DEPTH)
        def _():
            pltpu.make_async_copy(g_hbm.at[fidx.at[pl.ds(0, CH)]],
                                  dots.at[pl.ds(0, CH)], sem).wait()

        return carry

    lax.fori_loop(0, NCHUNK2, fire, 0)
    for _q in range(DEPTH):
        pltpu.make_async_copy(g_hbm.at[fidx.at[pl.ds(0, CH)]],
                              dots.at[pl.ds(0, CH)], sem).wait()

    def decode(t, carry):
        for q in range(5):
            o = (t * 5 + q) * 16
            w = plsc.bitcast(dots[pl.ds(o, 16)], jnp.int32)
            jv = jidx[pl.ds(o, 16)]
            half = (lax.shift_right_logical(jv, 7) & 1) == 1
            hi = w & jnp.int32(-65536)
            lo = lax.shift_left(w, 16)
            dots[pl.ds(o, 16)] = plsc.bitcast(
                jnp.where(half, hi, lo), jnp.float32)
        return carry

    lax.fori_loop(0, E2PT // 80, decode, 0)
    pltpu.sync_copy(dots, out_hbm.at[pl.ds(base, E2PT)])


NJB = NPAD // (2 * D)


def _dinv_from(deg_ref):
    deg = jnp.sum(deg_ref[...], axis=0)[:N]
    return lax.rsqrt(deg + 1.0)[:, None]


def _stage1_tc(deg_ref, x_ref, w1_ref, p1_ref):
    dinv = _dinv_from(deg_ref)
    g = jnp.dot(x_ref[...], w1_ref[...], preferred_element_type=jnp.float32,
                precision=lax.Precision.HIGHEST)
    p1_ref[...] = g * dinv


def _stage2_tc(deg_ref, s1_ref, p1_ref, b1_ref, w2_ref, p2_ref):
    dinv = _dinv_from(deg_ref)
    s1 = s1_ref[0][:N] + s1_ref[1][:N]
    h1 = jnp.maximum(dinv * (s1 + p1_ref[...]) + b1_ref[...], 0.0)
    g = jnp.dot(h1, w2_ref[...], preferred_element_type=jnp.float32,
                precision=lax.Precision.HIGHEST)
    p2_ref[...] = g * dinv


def _gram_fused_tc(deg_ref, s2_ref, p2_ref, b2_ref, g_ref, h2b):
    @pl.when(pl.program_id(0) == 0)
    def _():
        dinv = _dinv_from(deg_ref)
        s2 = s2_ref[0][:N] + s2_ref[1][:N]
        h2 = dinv * (s2 + p2_ref[...]) + b2_ref[...]
        h2b[...] = jnp.concatenate(
            [h2, jnp.zeros((NPAD - N, D), jnp.float32)],
            axis=0).astype(jnp.bfloat16)

    j = pl.program_id(0)
    a = h2b[...]
    b = h2b[pl.ds(j * (2 * D), 2 * D), :]
    res = lax.dot_general(a, b, (((1,), (1,)), ((), ())),
                          preferred_element_type=jnp.float32)
    ib = lax.bitcast_convert_type(res, jnp.int32)
    rnd = ib + 0x7FFF + (lax.shift_right_logical(ib, 16) & 1)
    b16 = lax.shift_right_logical(rnd, 16)
    packed = lax.shift_left(b16[:, D:], 16) | b16[:, :D]
    g_ref[...] = lax.bitcast_convert_type(packed, jnp.float32)[None]


_f32 = jnp.float32


def kernel(x, pos_edge_index, neg_edge_index, W1, b1, W2, b2):
    src = pos_edge_index[0]
    dst = pos_edge_index[1]

    zeros_d = jnp.zeros((NPT, D), _f32)

    deg_part = _degree_sc(dst).reshape(NW, NPAD)

    p1 = pl.pallas_call(
        _stage1_tc, out_shape=jax.ShapeDtypeStruct((N, D), _f32),
    )(deg_part, x, W1)

    s1 = _scatter_sc(p1, src, dst, zeros_d)

    p2 = pl.pallas_call(
        _stage2_tc, out_shape=jax.ShapeDtypeStruct((N, D), _f32),
    )(deg_part, s1, p1, b1, W2)

    s2 = _scatter_sc(p2, src, dst, zeros_d)

    gram = pl.pallas_call(
        _gram_fused_tc,
        grid=(NJB,),
        in_specs=[
            pl.BlockSpec((NW, NPAD), lambda j: (0, 0)),
            pl.BlockSpec((NC, NPAD, D), lambda j: (0, 0, 0)),
            pl.BlockSpec((N, D), lambda j: (0, 0)),
            pl.BlockSpec((D,), lambda j: (0,)),
        ],
        out_specs=pl.BlockSpec((1, NPAD, D), lambda j: (j, 0, 0)),
        out_shape=jax.ShapeDtypeStruct((NJB, NPAD, D), _f32),
        scratch_shapes=[pltpu.VMEM((NPAD, D), jnp.bfloat16)],
    )(deg_part, s2, p2, b2)

    ei = jnp.concatenate([pos_edge_index[0], neg_edge_index[0]])
    ej = jnp.concatenate([pos_edge_index[1], neg_edge_index[1]])
    return _link_sc(gram.reshape(-1), ei, ej)


# speedup vs baseline: 36.1759x; 1.0043x over previous
import functools

import jax
import jax.numpy as jnp
from jax import lax
from jax.experimental import pallas as pl
from jax.experimental.pallas import tpu as pltpu
from jax.experimental.pallas import tpu_sc as plsc

N = 10000
D = 128
E = 320000
E2 = 2 * E

NC = 2
NS = 16
NPAD = 10240
NPT = NPAD // NS

CH = 80
EPC = E // NC
EPT = EPC // NS
NCHUNK = EPT // CH

E2PC = E2 // NC
E2PT = E2PC // NS
NCHUNK2 = E2PT // CH

_mesh = plsc.VectorSubcoreMesh(core_axis_name="c", subcore_axis_name="s")


NW = NC * NS


@functools.partial(
    pl.kernel,
    out_type=jax.ShapeDtypeStruct((NW * NPAD,), jnp.float32),
    mesh=_mesh,
    compiler_params=pltpu.CompilerParams(needs_layout_passes=False),
    scratch_types=[
        pltpu.VMEM((NPAD,), jnp.float32),
        pltpu.VMEM((EPT,), jnp.int32),
    ],
)
def _degree_sc(dst_hbm, out_hbm, hist, didx):
    c = lax.axis_index("c")
    s = lax.axis_index("s")

    def z(r, carry):
        for q in range(16):
            hist[pl.ds((r * 16 + q) * 16, 16)] = jnp.zeros((16,), jnp.float32)
        return carry

    lax.fori_loop(0, NPAD // 256, z, 0)

    base = c * EPC + s * EPT
    pltpu.sync_copy(dst_hbm.at[pl.ds(base, EPT)], didx)
    ones = jnp.ones((16,), jnp.float32)

    def step(t, carry):
        for q in range(25):
            iv = didx[pl.ds((t * 25 + q) * 16, 16)]
            plsc.addupdate_scatter(hist, [iv], ones)
        return carry

    lax.fori_loop(0, EPT // (25 * 16), step, 0)
    wid = c * NS + s
    pltpu.sync_copy(hist, out_hbm.at[pl.ds(wid * NPAD, NPAD)])


NBUF = 3


@functools.partial(
    pl.kernel,
    out_type=jax.ShapeDtypeStruct((NC, NPAD, D), jnp.float32),
    mesh=_mesh,
    compiler_params=pltpu.CompilerParams(needs_layout_passes=False),
    scratch_types=[
        pltpu.VMEM_SHARED((NPAD, D), jnp.float32),
        pltpu.VMEM((EPT,), jnp.int32),
        pltpu.VMEM((NBUF, CH), jnp.int32),
        pltpu.VMEM((NBUF, CH, D), jnp.float32),
        pltpu.SemaphoreType.DMA((NBUF,)),
        pltpu.SemaphoreType.DMA((NBUF,)),
        pltpu.SemaphoreType.DMA((NBUF,)),
    ],
)
def _scatter_sc(table_hbm, src_hbm, dst_hbm, zeros_hbm, out_hbm, acc,
                sidx, dcur, rows, semg, semd, sems):
    c = lax.axis_index("c")
    s = lax.axis_index("s")
    pltpu.sync_copy(zeros_hbm, acc.at[pl.ds(s * NPT, NPT)])
    base = c * EPC + s * EPT
    pltpu.sync_copy(src_hbm.at[pl.ds(base, EPT)], sidx)
    plsc.subcore_barrier()

    for p in range(NBUF - 1):
        pltpu.async_copy(table_hbm.at[sidx.at[pl.ds(p * CH, CH)]],
                         rows.at[p], semg.at[p])
        pltpu.async_copy(dst_hbm.at[pl.ds(base + p * CH, CH)],
                         dcur.at[p], semd.at[p])

    def chunk(t, carry):
        b = lax.rem(t, NBUF)
        nb = lax.rem(t + NBUF - 1, NBUF)
        pltpu.make_async_copy(table_hbm.at[sidx.at[pl.ds(t * CH, CH)]],
                              rows.at[b], semg.at[b]).wait()
        pltpu.make_async_copy(dst_hbm.at[pl.ds(base, CH)],
                              dcur.at[b], semd.at[b]).wait()
        pltpu.async_copy(rows.at[b], acc.at[dcur.at[b]], sems.at[b], add=True)

        @pl.when(t + NBUF - 1 < NCHUNK)
        def _():
            @pl.when(t >= 1)
            def _():
                pltpu.make_async_copy(rows.at[nb], acc.at[dcur.at[nb]],
                                      sems.at[nb]).wait()
            pltpu.async_copy(
                table_hbm.at[sidx.at[pl.ds((t + NBUF - 1) * CH, CH)]],
                rows.at[nb], semg.at[nb])
            pltpu.async_copy(
                dst_hbm.at[pl.ds(base + (t + NBUF - 1) * CH, CH)],
                dcur.at[nb], semd.at[nb])

        return carry

    lax.fori_loop(0, NCHUNK, chunk, 0)
    for p in range(NBUF):
        pltpu.make_async_copy(rows.at[p], acc.at[dcur.at[p]],
                              sems.at[p]).wait()
    plsc.subcore_barrier()
    pltpu.sync_copy(acc.at[pl.ds(s * NPT, NPT)],
                    out_hbm.at[c, pl.ds(s * NPT, NPT)])


DEPTH = 24


@functools.partial(
    pl.kernel,
    out_type=jax.ShapeDtypeStruct((E2,), jnp.float32),
    mesh=_mesh,
    compiler_params=pltpu.CompilerParams(needs_layout_passes=False),
    scratch_types=[
        pltpu.VMEM((E2PT,), jnp.int32),
        pltpu.VMEM((E2PT,), jnp.int32),
        pltpu.VMEM((E2PT,), jnp.int32),
        pltpu.VMEM((E2PT,), jnp.float32),
        pltpu.SemaphoreType.DMA,
    ],
)
def _link_sc(g_hbm, ei_hbm, ej_hbm, out_hbm, iidx, jidx, fidx, dots, sem):
    c = lax.axis_index("c")
    s = lax.axis_index("s")
    base = c * E2PC + s * E2PT
    pltpu.sync_copy(ei_hbm.at[pl.ds(base, E2PT)], iidx)
    pltpu.sync_copy(ej_hbm.at[pl.ds(base, E2PT)], jidx)

    def flat(t, carry):
        for q in range(5):
            o = (t * 5 + q) * 16
            iv = iidx[pl.ds(o, 16)]
            jv = jidx[pl.ds(o, 16)]
            jb = lax.shift_right_logical(jv, 8)
            cc = jv & 127
            fidx[pl.ds(o, 16)] = jb * (NPAD * D) + iv * D + cc
        return carry

    lax.fori_loop(0, E2PT // 80, flat, 0)

    def fire(t, carry):
        pltpu.async_copy(g_hbm.at[fidx.at[pl.ds(t * CH, CH)]],
                         dots.at[pl.ds(t * CH, CH)], sem)

        @pl.when(t >= DEPTH)
        def _():
            pltpu.make_async_copy(g_hbm.at[fidx.at[pl.ds(0, CH)]],
                                  dots.at[pl.ds(0, CH)], sem).wait()

        return carry

    lax.fori_loop(0, NCHUNK2, fire, 0)
    for _q in range(DEPTH):
        pltpu.make_async_copy(g_hbm.at[fidx.at[pl.ds(0, CH)]],
                              dots.at[pl.ds(0, CH)], sem).wait()

    def decode(t, carry):
        for q in range(5):
            o = (t * 5 + q) * 16
            w = plsc.bitcast(dots[pl.ds(o, 16)], jnp.int32)
            jv = jidx[pl.ds(o, 16)]
            half = (lax.shift_right_logical(jv, 7) & 1) == 1
            hi = w & jnp.int32(-65536)
            lo = lax.shift_left(w, 16)
            dots[pl.ds(o, 16)] = plsc.bitcast(
                jnp.where(half, hi, lo), jnp.float32)
        return carry

    lax.fori_loop(0, E2PT // 80, decode, 0)
    pltpu.sync_copy(dots, out_hbm.at[pl.ds(base, E2PT)])


NJB = NPAD // (2 * D)


def _dinv_from(deg_ref):
    deg = jnp.sum(deg_ref[...], axis=0)[:N]
    return lax.rsqrt(deg + 1.0)[:, None]


def _stage1_tc(deg_ref, x_ref, w1_ref, p1_ref):
    dinv = _dinv_from(deg_ref)
    g = jnp.dot(x_ref[...], w1_ref[...], preferred_element_type=jnp.float32,
                precision=lax.Precision.HIGHEST)
    p1_ref[...] = g * dinv


def _stage2_tc(deg_ref, s1_ref, p1_ref, b1_ref, w2_ref, p2_ref):
    dinv = _dinv_from(deg_ref)
    s1 = s1_ref[0][:N] + s1_ref[1][:N]
    h1 = jnp.maximum(dinv * (s1 + p1_ref[...]) + b1_ref[...], 0.0)
    g = jnp.dot(h1, w2_ref[...], preferred_element_type=jnp.float32,
                precision=lax.Precision.HIGHEST)
    p2_ref[...] = g * dinv


def _gram_fused_tc(deg_ref, s2_ref, p2_ref, b2_ref, g_ref, h2b):
    @pl.when(pl.program_id(0) == 0)
    def _():
        dinv = _dinv_from(deg_ref)
        s2 = s2_ref[0][:N] + s2_ref[1][:N]
        h2 = dinv * (s2 + p2_ref[...]) + b2_ref[...]
        h2b[...] = jnp.concatenate(
            [h2, jnp.zeros((NPAD - N, D), jnp.float32)],
            axis=0).astype(jnp.bfloat16)

    j = pl.program_id(0)
    a = h2b[...]
    b = h2b[pl.ds(j * (2 * D), 2 * D), :]
    res = lax.dot_general(a, b, (((1,), (1,)), ((), ())),
                          preferred_element_type=jnp.float32)
    ib = lax.bitcast_convert_type(res, jnp.int32)
    rnd = ib + 0x7FFF + (lax.shift_right_logical(ib, 16) & 1)
    b16 = lax.shift_right_logical(rnd, 16)
    packed = lax.shift_left(b16[:, D:], 16) | b16[:, :D]
    g_ref[...] = lax.bitcast_convert_type(packed, jnp.float32)[None]


_f32 = jnp.float32


def kernel(x, pos_edge_index, neg_edge_index, W1, b1, W2, b2):
    src = pos_edge_index[0]
    dst = pos_edge_index[1]

    zeros_d = jnp.zeros((NPT, D), _f32)

    deg_part = _degree_sc(dst).reshape(NW, NPAD)

    p1 = pl.pallas_call(
        _stage1_tc, out_shape=jax.ShapeDtypeStruct((N, D), _f32),
    )(deg_part, x, W1)

    s1 = _scatter_sc(p1, src, dst, zeros_d)

    p2 = pl.pallas_call(
        _stage2_tc, out_shape=jax.ShapeDtypeStruct((N, D), _f32),
    )(deg_part, s1, p1, b1, W2)

    s2 = _scatter_sc(p2, src, dst, zeros_d)

    gram = pl.pallas_call(
        _gram_fused_tc,
        grid=(NJB,),
        in_specs=[
            pl.BlockSpec((NW, NPAD), lambda j: (0, 0)),
            pl.BlockSpec((NC, NPAD, D), lambda j: (0, 0, 0)),
            pl.BlockSpec((N, D), lambda j: (0, 0)),
            pl.BlockSpec((D,), lambda j: (0,)),
        ],
        out_specs=pl.BlockSpec((1, NPAD, D), lambda j: (j, 0, 0)),
        out_shape=jax.ShapeDtypeStruct((NJB, NPAD, D), _f32),
        scratch_shapes=[pltpu.VMEM((NPAD, D), jnp.bfloat16)],
    )(deg_part, s2, p2, b2)

    ei = jnp.concatenate([pos_edge_index[0], neg_edge_index[0]])
    ej = jnp.concatenate([pos_edge_index[1], neg_edge_index[1]])
    return _link_sc(gram.reshape(-1), ei, ej)
